# Initial kernel scaffold; baseline (speedup 1.0000x reference)
import jax
import jax.numpy as jnp
from jax.experimental import pallas as pl


def kernel(x, edge_attr, edge_index, We1_e, We1_s, be1, Wn1_n, Wn1_i, bn1, We2_e, We2_s, be2, Wn2_n, Wn2_i, bn2, We3_e, We3_s, be3, Wn3_n, Wn3_i, bn3, Wg_n, Wg_e, bg):
    raise NotImplementedError("write your pallas kernel here")


# speedup vs baseline: 1.6002x; 1.6002x over previous
import functools

import jax
import jax.numpy as jnp
from jax import lax
from jax.experimental import pallas as pl
from jax.experimental.pallas import tpu as pltpu
from jax.experimental.pallas import tpu_sc as plsc

NC = 2
NS = 16
NW = NC * NS

_N = 10000
_E = 320000


def _make_sc_gather(V, D, E):
    ew = E // NW
    C = 80
    iters = ew // C
    assert ew % C == 0 and C % 8 == 0

    mesh = plsc.VectorSubcoreMesh(core_axis_name="c", subcore_axis_name="s")

    @functools.partial(
        pl.kernel,
        mesh=mesh,
        out_type=jax.ShapeDtypeStruct((E, D), jnp.float32),
        scratch_types=[
            pltpu.VMEM((C,), jnp.int32),
            pltpu.VMEM((C, D), jnp.float32),
            pltpu.SemaphoreType.DMA,
        ],
        compiler_params=pltpu.CompilerParams(use_tc_tiling_on_sc=False),
    )
    def k(table_hbm, idx_hbm, out_hbm, idx_v, rows_v, sem):
        wid = lax.axis_index("s") * NC + lax.axis_index("c")
        base = wid * ew

        def body(j, _):
            off = base + j * C
            pltpu.sync_copy(idx_hbm.at[pl.ds(off, C)], idx_v)
            pltpu.async_copy(table_hbm.at[idx_v], rows_v, sem).wait()
            pltpu.sync_copy(rows_v, out_hbm.at[pl.ds(off, C), :])
            return 0

        lax.fori_loop(0, iters, body, 0)

    return k


def _make_sc_segsum(E, N, D2, with_counts):
    ew = E // NS
    C = 80
    iters = ew // C
    rows_pt = N // NS
    ZR = 25
    assert ew % C == 0 and rows_pt % ZR == 0

    mesh = plsc.VectorSubcoreMesh(core_axis_name="c", subcore_axis_name="s")

    out_type = [
        jax.ShapeDtypeStruct((N, D2), jnp.float32),
        jax.ShapeDtypeStruct((N, D2), jnp.float32),
    ]
    scratch = [
        pltpu.VMEM_SHARED((N, D2), jnp.float32),
        pltpu.VMEM((C, D2), jnp.float32),
        pltpu.VMEM((C,), jnp.int32),
        pltpu.VMEM((ZR, D2), jnp.float32),
    ]
    if with_counts:
        out_type.append(jax.ShapeDtypeStruct((N, 16), jnp.float32))
        scratch += [
            pltpu.VMEM_SHARED((N, 16), jnp.float32),
            pltpu.VMEM((C, 16), jnp.float32),
            pltpu.VMEM((ZR, 16), jnp.float32),
        ]

    def body(*refs):
        if with_counts:
            (e_lo, e_hi, recv, out_lo, out_hi, out_cnt,
             acc, ebuf, idx_v, zbuf, cacc, ones_v, zc) = refs
        else:
            (e_lo, e_hi, recv, out_lo, out_hi,
             acc, ebuf, idx_v, zbuf) = refs
        cid = lax.axis_index("c")
        tid = lax.axis_index("s")

        zero16 = jnp.zeros((16,), jnp.float32)
        for r in range(ZR):
            for q in range(D2 // 16):
                zbuf[r, pl.ds(q * 16, 16)] = zero16
        if with_counts:
            one16 = jnp.ones((16,), jnp.float32)
            for r in range(ZR):
                zc[r, pl.ds(0, 16)] = zero16
            for r in range(C):
                ones_v[r, pl.ds(0, 16)] = one16

        def zinit(j, _):
            r0 = tid * rows_pt + j * ZR
            pltpu.sync_copy(zbuf, acc.at[pl.ds(r0, ZR), :])
            if with_counts:
                pltpu.sync_copy(zc, cacc.at[pl.ds(r0, ZR), :])
            return 0

        lax.fori_loop(0, rows_pt // ZR, zinit, 0)
        plsc.subcore_barrier()

        def chunk(j, _):
            off = tid * ew + j * C
            pltpu.sync_copy(recv.at[pl.ds(off, C)], idx_v)

            @pl.when(cid == 0)
            def _():
                pltpu.sync_copy(e_lo.at[pl.ds(off, C), :], ebuf)

            @pl.when(cid == 1)
            def _():
                pltpu.sync_copy(e_hi.at[pl.ds(off, C), :], ebuf)

            pltpu.sync_copy(ebuf, acc.at[idx_v], add=True)
            if with_counts:
                @pl.when(cid == 0)
                def _():
                    pltpu.sync_copy(ones_v, cacc.at[idx_v], add=True)
            return 0

        lax.fori_loop(0, iters, chunk, 0)
        plsc.subcore_barrier()

        def drain(j, _):
            r0 = tid * rows_pt + j * ZR
            pltpu.sync_copy(acc.at[pl.ds(r0, ZR), :], zbuf)

            @pl.when(cid == 0)
            def _():
                pltpu.sync_copy(zbuf, out_lo.at[pl.ds(r0, ZR), :])

            @pl.when(cid == 1)
            def _():
                pltpu.sync_copy(zbuf, out_hi.at[pl.ds(r0, ZR), :])

            if with_counts:
                @pl.when(cid == 0)
                def _():
                    pltpu.sync_copy(cacc.at[pl.ds(r0, ZR), :], zc)
                    pltpu.sync_copy(zc, out_cnt.at[pl.ds(r0, ZR), :])
            return 0

        lax.fori_loop(0, rows_pt // ZR, drain, 0)

    return functools.partial(
        pl.kernel, mesh=mesh, out_type=tuple(out_type),
        scratch_types=scratch,
        compiler_params=pltpu.CompilerParams(use_tc_tiling_on_sc=False),
    )(body)


_BE = 1000


def _edge1_body(ea_ref, xs_ref, we_ref, ws_ref, b_ref, lo_ref, hi_ref):
    acc = jnp.dot(ea_ref[...], we_ref[...], preferred_element_type=jnp.float32)
    acc += jnp.dot(xs_ref[...], ws_ref[...], preferred_element_type=jnp.float32)
    e = jnp.maximum(acc + b_ref[...], 0.0)
    lo_ref[...] = e[:, :128]
    hi_ref[...] = e[:, 128:]


def _tc_edge1(edge_attr, xs, We1_e, We1_s_pad, be1):
    grid = _E // _BE
    return pl.pallas_call(
        _edge1_body,
        grid=(grid,),
        in_specs=[
            pl.BlockSpec((_BE, 10), lambda i: (i, 0)),
            pl.BlockSpec((_BE, 16), lambda i: (i, 0)),
            pl.BlockSpec((10, 256), lambda i: (0, 0)),
            pl.BlockSpec((16, 256), lambda i: (0, 0)),
            pl.BlockSpec((1, 256), lambda i: (0, 0)),
        ],
        out_specs=[
            pl.BlockSpec((_BE, 128), lambda i: (i, 0)),
            pl.BlockSpec((_BE, 128), lambda i: (i, 0)),
        ],
        out_shape=[
            jax.ShapeDtypeStruct((_E, 128), jnp.float32),
            jax.ShapeDtypeStruct((_E, 128), jnp.float32),
        ],
    )(edge_attr, xs, We1_e, We1_s_pad, be1.reshape(1, 256))


def _edge_mid_body(lo_ref, hi_ref, g_ref, w_ref, b_ref, olo_ref, ohi_ref):
    d_in2, d_out = w_ref.shape
    d_in = d_in2 // 2
    acc = jnp.dot(lo_ref[...], w_ref[:d_in, :], preferred_element_type=jnp.float32)
    acc += jnp.dot(hi_ref[...], w_ref[d_in:, :], preferred_element_type=jnp.float32)
    e = jnp.maximum(acc + g_ref[...] + b_ref[...], 0.0)
    h = d_out // 2
    olo_ref[...] = e[:, :h]
    ohi_ref[...] = e[:, h:]


def _tc_edge2(e1_lo, e1_hi, g2, We2_e, be2):
    grid = _E // _BE
    return pl.pallas_call(
        _edge_mid_body,
        grid=(grid,),
        in_specs=[
            pl.BlockSpec((_BE, 128), lambda i: (i, 0)),
            pl.BlockSpec((_BE, 128), lambda i: (i, 0)),
            pl.BlockSpec((_BE, 128), lambda i: (i, 0)),
            pl.BlockSpec((256, 128), lambda i: (0, 0)),
            pl.BlockSpec((1, 128), lambda i: (0, 0)),
        ],
        out_specs=[
            pl.BlockSpec((_BE, 64), lambda i: (i, 0)),
            pl.BlockSpec((_BE, 64), lambda i: (i, 0)),
        ],
        out_shape=[
            jax.ShapeDtypeStruct((_E, 64), jnp.float32),
            jax.ShapeDtypeStruct((_E, 64), jnp.float32),
        ],
    )(e1_lo, e1_hi, g2, We2_e, be2.reshape(1, 128))


def _edge3_body(lo_ref, hi_ref, g_ref, w_ref, b_ref, olo_ref, ohi_ref, sum_ref):
    i = pl.program_id(0)
    acc = jnp.dot(lo_ref[...], w_ref[:64, :], preferred_element_type=jnp.float32)
    acc += jnp.dot(hi_ref[...], w_ref[64:, :], preferred_element_type=jnp.float32)
    e = jnp.maximum(acc + g_ref[...] + b_ref[...], 0.0)
    olo_ref[...] = e[:, :32]
    ohi_ref[...] = e[:, 32:]

    @pl.when(i == 0)
    def _():
        sum_ref[...] = jnp.zeros_like(sum_ref)

    sum_ref[...] += jnp.sum(e.reshape(_BE // 8, 8, 64), axis=0)


def _tc_edge3(e2_lo, e2_hi, g3, We3_e, be3):
    grid = _E // _BE
    return pl.pallas_call(
        _edge3_body,
        grid=(grid,),
        in_specs=[
            pl.BlockSpec((_BE, 64), lambda i: (i, 0)),
            pl.BlockSpec((_BE, 64), lambda i: (i, 0)),
            pl.BlockSpec((_BE, 64), lambda i: (i, 0)),
            pl.BlockSpec((128, 64), lambda i: (0, 0)),
            pl.BlockSpec((1, 64), lambda i: (0, 0)),
        ],
        out_specs=[
            pl.BlockSpec((_BE, 32), lambda i: (i, 0)),
            pl.BlockSpec((_BE, 32), lambda i: (i, 0)),
            pl.BlockSpec((8, 64), lambda i: (0, 0)),
        ],
        out_shape=[
            jax.ShapeDtypeStruct((_E, 32), jnp.float32),
            jax.ShapeDtypeStruct((_E, 32), jnp.float32),
            jax.ShapeDtypeStruct((8, 64), jnp.float32),
        ],
    )(e2_lo, e2_hi, g3, We3_e, be3.reshape(1, 64))


_BN = 1000


def _node_body(prev_ref, slo_ref, shi_ref, cnt_ref, wn_ref, wi_ref, b_ref,
               ws_ref, n_ref, m_ref):
    cnt = jnp.maximum(cnt_ref[:, 0:1], 1.0)
    s = jnp.concatenate([slo_ref[...], shi_ref[...]], axis=1) / cnt
    acc = jnp.dot(prev_ref[...], wn_ref[...], preferred_element_type=jnp.float32)
    acc += jnp.dot(s, wi_ref[...], preferred_element_type=jnp.float32)
    n = jnp.maximum(acc + b_ref[...], 0.0)
    n_ref[...] = n
    m_ref[...] = jnp.dot(n, ws_ref[...], preferred_element_type=jnp.float32)


def _tc_node(prev, s_lo, s_hi, cnt, Wn, Wi, b, Ws):
    d_prev = prev.shape[1]
    d2 = s_lo.shape[1]
    d_out = Wn.shape[1]
    d_m = Ws.shape[1]
    grid = _N // _BN
    return pl.pallas_call(
        _node_body,
        grid=(grid,),
        in_specs=[
            pl.BlockSpec((_BN, d_prev), lambda i: (i, 0)),
            pl.BlockSpec((_BN, d2), lambda i: (i, 0)),
            pl.BlockSpec((_BN, d2), lambda i: (i, 0)),
            pl.BlockSpec((_BN, 16), lambda i: (i, 0)),
            pl.BlockSpec((d_prev, d_out), lambda i: (0, 0)),
            pl.BlockSpec((2 * d2, d_out), lambda i: (0, 0)),
            pl.BlockSpec((1, d_out), lambda i: (0, 0)),
            pl.BlockSpec((d_out, d_m), lambda i: (0, 0)),
        ],
        out_specs=[
            pl.BlockSpec((_BN, d_out), lambda i: (i, 0)),
            pl.BlockSpec((_BN, d_m), lambda i: (i, 0)),
        ],
        out_shape=[
            jax.ShapeDtypeStruct((_N, d_out), jnp.float32),
            jax.ShapeDtypeStruct((_N, d_m), jnp.float32),
        ],
    )(prev, s_lo, s_hi, cnt, Wn, Wi, b.reshape(1, d_out), Ws)


def _node3_body(prev_ref, slo_ref, shi_ref, cnt_ref, esum_ref, wn_ref, wi_ref,
                b_ref, wgn_ref, wge_ref, bg_ref, g_ref, acc_ref):
    i = pl.program_id(0)
    cnt = jnp.maximum(cnt_ref[:, 0:1], 1.0)
    s = jnp.concatenate([slo_ref[...], shi_ref[...]], axis=1) / cnt
    acc = jnp.dot(prev_ref[...], wn_ref[...], preferred_element_type=jnp.float32)
    acc += jnp.dot(s, wi_ref[...], preferred_element_type=jnp.float32)
    n3 = jnp.maximum(acc + b_ref[...], 0.0)

    @pl.when(i == 0)
    def _():
        acc_ref[...] = jnp.zeros_like(acc_ref)

    acc_ref[...] += jnp.sum(n3.reshape(_BN // 8, 8, 64), axis=0)

    nmean = jnp.sum(acc_ref[...], axis=0, keepdims=True) * (1.0 / _N)
    emean = jnp.sum(esum_ref[...], axis=0, keepdims=True) * (1.0 / _E)
    g = jnp.dot(nmean, wgn_ref[...], preferred_element_type=jnp.float32)
    g += jnp.dot(emean, wge_ref[...], preferred_element_type=jnp.float32)
    g_ref[...] = g + bg_ref[...]


def _tc_node3(n2, s_lo, s_hi, cnt, e3sum, Wn3_n, Wn3_i, bn3, Wg_n, Wg_e, bg):
    grid = _N // _BN
    return pl.pallas_call(
        _node3_body,
        grid=(grid,),
        in_specs=[
            pl.BlockSpec((_BN, 128), lambda i: (i, 0)),
            pl.BlockSpec((_BN, 32), lambda i: (i, 0)),
            pl.BlockSpec((_BN, 32), lambda i: (i, 0)),
            pl.BlockSpec((_BN, 16), lambda i: (i, 0)),
            pl.BlockSpec((8, 64), lambda i: (0, 0)),
            pl.BlockSpec((128, 64), lambda i: (0, 0)),
            pl.BlockSpec((64, 64), lambda i: (0, 0)),
            pl.BlockSpec((1, 64), lambda i: (0, 0)),
            pl.BlockSpec((64, 128), lambda i: (0, 0)),
            pl.BlockSpec((64, 128), lambda i: (0, 0)),
            pl.BlockSpec((1, 128), lambda i: (0, 0)),
        ],
        out_specs=pl.BlockSpec((1, 128), lambda i: (0, 0)),
        out_shape=jax.ShapeDtypeStruct((1, 128), jnp.float32),
        scratch_shapes=[pltpu.VMEM((8, 64), jnp.float32)],
    )(n2, s_lo, s_hi, cnt, e3sum, Wn3_n, Wn3_i, bn3.reshape(1, 64),
      Wg_n, Wg_e, bg.reshape(1, 128))


_sc_gather16 = _make_sc_gather(_N, 16, _E)
_sc_gather128 = _make_sc_gather(_N, 128, _E)
_sc_gather64 = _make_sc_gather(_N, 64, _E)
_sc_segsum128c = _make_sc_segsum(_E, _N, 128, True)
_sc_segsum64 = _make_sc_segsum(_E, _N, 64, False)
_sc_segsum32 = _make_sc_segsum(_E, _N, 32, False)


def kernel(x, edge_attr, edge_index, We1_e, We1_s, be1, Wn1_n, Wn1_i, bn1,
           We2_e, We2_s, be2, Wn2_n, Wn2_i, bn2, We3_e, We3_s, be3,
           Wn3_n, Wn3_i, bn3, Wg_n, Wg_e, bg):
    senders = edge_index[0]
    receivers = edge_index[1]
    x_pad = jnp.pad(x, ((0, 0), (0, 2)))
    We1_s_pad = jnp.pad(We1_s, ((0, 2), (0, 0)))

    xs = _sc_gather16(x_pad, senders)
    e1_lo, e1_hi = _tc_edge1(edge_attr, xs, We1_e, We1_s_pad, be1)
    s1_lo, s1_hi, cnt = _sc_segsum128c(e1_lo, e1_hi, receivers)
    n1, m1 = _tc_node(x, s1_lo, s1_hi, cnt, Wn1_n, Wn1_i, bn1, We2_s)

    g2 = _sc_gather128(m1, senders)
    e2_lo, e2_hi = _tc_edge2(e1_lo, e1_hi, g2, We2_e, be2)
    s2_lo, s2_hi = _sc_segsum64(e2_lo, e2_hi, receivers)
    n2, m2 = _tc_node(n1, s2_lo, s2_hi, cnt, Wn2_n, Wn2_i, bn2, We3_s)

    g3 = _sc_gather64(m2, senders)
    e3_lo, e3_hi, e3sum = _tc_edge3(e2_lo, e2_hi, g3, We3_e, be3)
    s3_lo, s3_hi = _sc_segsum32(e3_lo, e3_hi, receivers)
    g = _tc_node3(n2, s3_lo, s3_hi, cnt, e3sum, Wn3_n, Wn3_i, bn3,
                  Wg_n, Wg_e, bg)
    return g.reshape(128)


# speedup vs baseline: 1.9948x; 1.2466x over previous
import functools

import jax
import jax.numpy as jnp
from jax import lax
from jax.experimental import pallas as pl
from jax.experimental.pallas import tpu as pltpu
from jax.experimental.pallas import tpu_sc as plsc

NC = 2
NS = 16
NW = NC * NS

_N = 10000
_E = 320000


def _make_sc_gather(V, D, E, with_counts=False):
    ew = E // NW
    C = 80
    iters = ew // C
    rows_pt = _N // NS
    ZR = 25
    assert ew % C == 0 and C % 8 == 0

    mesh = plsc.VectorSubcoreMesh(core_axis_name="c", subcore_axis_name="s")

    out_type = [jax.ShapeDtypeStruct((E, D), jnp.float32)]
    scratch = [
        pltpu.VMEM((iters, C), jnp.int32),
        pltpu.VMEM((C, D), jnp.float32),
        pltpu.VMEM((C, D), jnp.float32),
        pltpu.SemaphoreType.DMA,
        pltpu.SemaphoreType.DMA,
        pltpu.SemaphoreType.DMA,
        pltpu.SemaphoreType.DMA,
    ]
    if with_counts:
        out_type += [jax.ShapeDtypeStruct((_N, 16), jnp.float32),
                     jax.ShapeDtypeStruct((_N, 16), jnp.float32)]
        scratch += [
            pltpu.VMEM((iters, C), jnp.int32),
            pltpu.VMEM_SHARED((_N, 16), jnp.float32),
            pltpu.VMEM((C, 16), jnp.float32),
            pltpu.SemaphoreType.DMA,
        ]

    def body(*refs):
        if with_counts:
            (table_hbm, idx_hbm, ridx_hbm, out_hbm, cnt_a, cnt_b,
             idx_all, rows0, rows1, g0, g1, w0, w1,
             ridx_all, cacc, ones_v, csem) = refs
        else:
            (table_hbm, idx_hbm, out_hbm,
             idx_all, rows0, rows1, g0, g1, w0, w1) = refs
        cid = lax.axis_index("c")
        tid = lax.axis_index("s")
        wid = tid * NC + cid
        base = wid * ew
        pltpu.sync_copy(idx_hbm.at[wid], idx_all)

        if with_counts:
            pltpu.sync_copy(ridx_hbm.at[wid], ridx_all)
            zero16 = jnp.zeros((16,), jnp.float32)
            one16 = jnp.ones((16,), jnp.float32)
            for r in range(ZR):
                rows0[r, pl.ds(0, 16)] = zero16
            for r in range(C):
                ones_v[r, pl.ds(0, 16)] = one16

            def zinit(j, _):
                r0 = tid * rows_pt + j * ZR
                pltpu.sync_copy(rows0.at[pl.ds(0, ZR), pl.ds(0, 16)],
                                cacc.at[pl.ds(r0, ZR), :])
                return 0

            lax.fori_loop(0, rows_pt // ZR, zinit, 0)
            plsc.subcore_barrier()

        pltpu.async_copy(table_hbm.at[idx_all.at[0]], rows0, g0)

        def phase(j, cur, nxt, gcur, gnxt, wcur, wnxt):
            pltpu.make_async_copy(table_hbm.at[idx_all.at[j]], cur, gcur).wait()
            pltpu.async_copy(cur, out_hbm.at[pl.ds(base + j * C, C), :], wcur)
            if with_counts:
                @pl.when(j >= 1)
                def _():
                    pltpu.make_async_copy(
                        ones_v, cacc.at[ridx_all.at[0]], csem).wait()
                pltpu.async_copy(ones_v, cacc.at[ridx_all.at[j]], csem,
                                 add=True)

            @pl.when(j + 1 < iters)
            def _():
                @pl.when(j >= 1)
                def _():
                    pltpu.make_async_copy(
                        nxt, out_hbm.at[pl.ds(base, C), :], wnxt).wait()
                pltpu.async_copy(table_hbm.at[idx_all.at[j + 1]], nxt, gnxt)

        def loop_body(j, _):
            @pl.when(j % 2 == 0)
            def _():
                phase(j, rows0, rows1, g0, g1, w0, w1)

            @pl.when(j % 2 == 1)
            def _():
                phase(j, rows1, rows0, g1, g0, w1, w0)
            return 0

        lax.fori_loop(0, iters, loop_body, 0)
        pltpu.make_async_copy(rows0, out_hbm.at[pl.ds(base, C), :], w0).wait()
        pltpu.make_async_copy(rows1, out_hbm.at[pl.ds(base, C), :], w1).wait()

        if with_counts:
            pltpu.make_async_copy(ones_v, cacc.at[ridx_all.at[0]], csem).wait()
            plsc.subcore_barrier()

            def drain(j, _):
                r0 = tid * rows_pt + j * ZR
                stg = rows0.at[pl.ds(0, ZR), pl.ds(0, 16)]
                pltpu.sync_copy(cacc.at[pl.ds(r0, ZR), :], stg)

                @pl.when(cid == 0)
                def _():
                    pltpu.sync_copy(stg, cnt_a.at[pl.ds(r0, ZR), :])

                @pl.when(cid == 1)
                def _():
                    pltpu.sync_copy(stg, cnt_b.at[pl.ds(r0, ZR), :])
                return 0

            lax.fori_loop(0, rows_pt // ZR, drain, 0)

    return functools.partial(
        pl.kernel, mesh=mesh, out_type=tuple(out_type) if with_counts
        else out_type[0],
        scratch_types=scratch,
        compiler_params=pltpu.CompilerParams(use_tc_tiling_on_sc=False),
    )(body)


def _make_sc_segsum(E, N, D2):
    ew = E // NS
    C = 80
    iters = ew // C
    rows_pt = N // NS
    ZR = 25
    assert ew % C == 0 and rows_pt % ZR == 0

    mesh = plsc.VectorSubcoreMesh(core_axis_name="c", subcore_axis_name="s")

    out_type = [
        jax.ShapeDtypeStruct((N, D2), jnp.float32),
        jax.ShapeDtypeStruct((N, D2), jnp.float32),
    ]
    scratch = [
        pltpu.VMEM_SHARED((N, D2), jnp.float32),
        pltpu.VMEM((C, D2), jnp.float32),
        pltpu.VMEM((C, D2), jnp.float32),
        pltpu.VMEM((iters, C), jnp.int32),
        pltpu.SemaphoreType.DMA,
        pltpu.SemaphoreType.DMA,
        pltpu.SemaphoreType.DMA,
        pltpu.SemaphoreType.DMA,
    ]

    def body(e_lo, e_hi, recv, out_lo, out_hi,
             acc, eb0, eb1, idx_all, r0s, r1s, s0s, s1s):
        cid = lax.axis_index("c")
        tid = lax.axis_index("s")
        base = tid * ew

        pltpu.sync_copy(recv.at[tid], idx_all)

        def rstart(j, buf, sem):
            @pl.when(cid == 0)
            def _():
                pltpu.async_copy(e_lo.at[pl.ds(base + j * C, C), :], buf, sem)

            @pl.when(cid == 1)
            def _():
                pltpu.async_copy(e_hi.at[pl.ds(base + j * C, C), :], buf, sem)

        def rwait(buf, sem):
            pltpu.make_async_copy(
                e_lo.at[pl.ds(base, C), :], buf, sem).wait()

        zero16 = jnp.zeros((16,), jnp.float32)
        for r in range(ZR):
            for q in range(D2 // 16):
                eb0[r, pl.ds(q * 16, 16)] = zero16

        def zinit(j, _):
            r0 = tid * rows_pt + j * ZR
            pltpu.sync_copy(eb0.at[pl.ds(0, ZR), :], acc.at[pl.ds(r0, ZR), :])
            return 0

        lax.fori_loop(0, rows_pt // ZR, zinit, 0)
        plsc.subcore_barrier()
        rstart(0, eb0, r0s)

        def swait(buf, sem):
            pltpu.make_async_copy(buf, acc.at[idx_all.at[0]], sem).wait()

        def phase(j, cur, nxt, rcur, rnxt, scur, snxt):
            rwait(cur, rcur)
            pltpu.async_copy(cur, acc.at[idx_all.at[j]], scur, add=True)

            @pl.when(j + 1 < iters)
            def _():
                @pl.when(j >= 1)
                def _():
                    swait(nxt, snxt)
                rstart(j + 1, nxt, rnxt)

        def chunk(j, _):
            @pl.when(j % 2 == 0)
            def _():
                phase(j, eb0, eb1, r0s, r1s, s0s, s1s)

            @pl.when(j % 2 == 1)
            def _():
                phase(j, eb1, eb0, r1s, r0s, s1s, s0s)
            return 0

        lax.fori_loop(0, iters, chunk, 0)
        swait(eb0, s0s)
        swait(eb1, s1s)
        plsc.subcore_barrier()

        def drain(j, _):
            r0 = tid * rows_pt + j * ZR
            pltpu.sync_copy(acc.at[pl.ds(r0, ZR), :], eb0.at[pl.ds(0, ZR), :])

            @pl.when(cid == 0)
            def _():
                pltpu.sync_copy(eb0.at[pl.ds(0, ZR), :],
                                out_lo.at[pl.ds(r0, ZR), :])

            @pl.when(cid == 1)
            def _():
                pltpu.sync_copy(eb0.at[pl.ds(0, ZR), :],
                                out_hi.at[pl.ds(r0, ZR), :])
            return 0

        lax.fori_loop(0, rows_pt // ZR, drain, 0)

    return functools.partial(
        pl.kernel, mesh=mesh, out_type=tuple(out_type),
        scratch_types=scratch,
        compiler_params=pltpu.CompilerParams(use_tc_tiling_on_sc=False),
    )(body)


_BE = 1000


def _edge1_body(ea_ref, xs_ref, we_ref, ws_ref, b_ref, lo_ref, hi_ref):
    acc = jnp.dot(ea_ref[...], we_ref[...], preferred_element_type=jnp.float32)
    acc += jnp.dot(xs_ref[...], ws_ref[...], preferred_element_type=jnp.float32)
    e = jnp.maximum(acc + b_ref[...], 0.0)
    lo_ref[...] = e[:, :128]
    hi_ref[...] = e[:, 128:]


def _tc_edge1(edge_attr, xs, We1_e, We1_s_pad, be1):
    grid = _E // _BE
    return pl.pallas_call(
        _edge1_body,
        grid=(grid,),
        in_specs=[
            pl.BlockSpec((_BE, 10), lambda i: (i, 0)),
            pl.BlockSpec((_BE, 16), lambda i: (i, 0)),
            pl.BlockSpec((10, 256), lambda i: (0, 0)),
            pl.BlockSpec((16, 256), lambda i: (0, 0)),
            pl.BlockSpec((1, 256), lambda i: (0, 0)),
        ],
        out_specs=[
            pl.BlockSpec((_BE, 128), lambda i: (i, 0)),
            pl.BlockSpec((_BE, 128), lambda i: (i, 0)),
        ],
        out_shape=[
            jax.ShapeDtypeStruct((_E, 128), jnp.float32),
            jax.ShapeDtypeStruct((_E, 128), jnp.float32),
        ],
    )(edge_attr, xs, We1_e, We1_s_pad, be1.reshape(1, 256))


def _edge_mid_body(lo_ref, hi_ref, g_ref, w_ref, b_ref, olo_ref, ohi_ref):
    d_in2, d_out = w_ref.shape
    d_in = d_in2 // 2
    acc = jnp.dot(lo_ref[...], w_ref[:d_in, :], preferred_element_type=jnp.float32)
    acc += jnp.dot(hi_ref[...], w_ref[d_in:, :], preferred_element_type=jnp.float32)
    e = jnp.maximum(acc + g_ref[...] + b_ref[...], 0.0)
    h = d_out // 2
    olo_ref[...] = e[:, :h]
    ohi_ref[...] = e[:, h:]


def _tc_edge2(e1_lo, e1_hi, g2, We2_e, be2):
    grid = _E // _BE
    return pl.pallas_call(
        _edge_mid_body,
        grid=(grid,),
        in_specs=[
            pl.BlockSpec((_BE, 128), lambda i: (i, 0)),
            pl.BlockSpec((_BE, 128), lambda i: (i, 0)),
            pl.BlockSpec((_BE, 128), lambda i: (i, 0)),
            pl.BlockSpec((256, 128), lambda i: (0, 0)),
            pl.BlockSpec((1, 128), lambda i: (0, 0)),
        ],
        out_specs=[
            pl.BlockSpec((_BE, 64), lambda i: (i, 0)),
            pl.BlockSpec((_BE, 64), lambda i: (i, 0)),
        ],
        out_shape=[
            jax.ShapeDtypeStruct((_E, 64), jnp.float32),
            jax.ShapeDtypeStruct((_E, 64), jnp.float32),
        ],
    )(e1_lo, e1_hi, g2, We2_e, be2.reshape(1, 128))


def _edge3_body(lo_ref, hi_ref, g_ref, w_ref, b_ref, olo_ref, ohi_ref, sum_ref):
    i = pl.program_id(0)
    acc = jnp.dot(lo_ref[...], w_ref[:64, :], preferred_element_type=jnp.float32)
    acc += jnp.dot(hi_ref[...], w_ref[64:, :], preferred_element_type=jnp.float32)
    e = jnp.maximum(acc + g_ref[...] + b_ref[...], 0.0)
    olo_ref[...] = e[:, :32]
    ohi_ref[...] = e[:, 32:]

    @pl.when(i == 0)
    def _():
        sum_ref[...] = jnp.zeros_like(sum_ref)

    sum_ref[...] += jnp.sum(e.reshape(_BE // 8, 8, 64), axis=0)


def _tc_edge3(e2_lo, e2_hi, g3, We3_e, be3):
    grid = _E // _BE
    return pl.pallas_call(
        _edge3_body,
        grid=(grid,),
        in_specs=[
            pl.BlockSpec((_BE, 64), lambda i: (i, 0)),
            pl.BlockSpec((_BE, 64), lambda i: (i, 0)),
            pl.BlockSpec((_BE, 64), lambda i: (i, 0)),
            pl.BlockSpec((128, 64), lambda i: (0, 0)),
            pl.BlockSpec((1, 64), lambda i: (0, 0)),
        ],
        out_specs=[
            pl.BlockSpec((_BE, 32), lambda i: (i, 0)),
            pl.BlockSpec((_BE, 32), lambda i: (i, 0)),
            pl.BlockSpec((8, 64), lambda i: (0, 0)),
        ],
        out_shape=[
            jax.ShapeDtypeStruct((_E, 32), jnp.float32),
            jax.ShapeDtypeStruct((_E, 32), jnp.float32),
            jax.ShapeDtypeStruct((8, 64), jnp.float32),
        ],
    )(e2_lo, e2_hi, g3, We3_e, be3.reshape(1, 64))


_BN = 1000


def _node_body(prev_ref, slo_ref, shi_ref, ca_ref, cb_ref, wn_ref, wi_ref,
               b_ref, ws_ref, n_ref, m_ref):
    cnt = jnp.maximum(ca_ref[:, 0:1] + cb_ref[:, 0:1], 1.0)
    s = jnp.concatenate([slo_ref[...], shi_ref[...]], axis=1) / cnt
    acc = jnp.dot(prev_ref[...], wn_ref[...], preferred_element_type=jnp.float32)
    acc += jnp.dot(s, wi_ref[...], preferred_element_type=jnp.float32)
    n = jnp.maximum(acc + b_ref[...], 0.0)
    n_ref[...] = n
    m_ref[...] = jnp.dot(n, ws_ref[...], preferred_element_type=jnp.float32)


def _tc_node(prev, s_lo, s_hi, cnt_a, cnt_b, Wn, Wi, b, Ws):
    d_prev = prev.shape[1]
    d2 = s_lo.shape[1]
    d_out = Wn.shape[1]
    d_m = Ws.shape[1]
    grid = _N // _BN
    return pl.pallas_call(
        _node_body,
        grid=(grid,),
        in_specs=[
            pl.BlockSpec((_BN, d_prev), lambda i: (i, 0)),
            pl.BlockSpec((_BN, d2), lambda i: (i, 0)),
            pl.BlockSpec((_BN, d2), lambda i: (i, 0)),
            pl.BlockSpec((_BN, 16), lambda i: (i, 0)),
            pl.BlockSpec((_BN, 16), lambda i: (i, 0)),
            pl.BlockSpec((d_prev, d_out), lambda i: (0, 0)),
            pl.BlockSpec((2 * d2, d_out), lambda i: (0, 0)),
            pl.BlockSpec((1, d_out), lambda i: (0, 0)),
            pl.BlockSpec((d_out, d_m), lambda i: (0, 0)),
        ],
        out_specs=[
            pl.BlockSpec((_BN, d_out), lambda i: (i, 0)),
            pl.BlockSpec((_BN, d_m), lambda i: (i, 0)),
        ],
        out_shape=[
            jax.ShapeDtypeStruct((_N, d_out), jnp.float32),
            jax.ShapeDtypeStruct((_N, d_m), jnp.float32),
        ],
    )(prev, s_lo, s_hi, cnt_a, cnt_b, Wn, Wi, b.reshape(1, d_out), Ws)


def _node3_body(prev_ref, slo_ref, shi_ref, ca_ref, cb_ref, esum_ref, wn_ref,
                wi_ref, b_ref, wgn_ref, wge_ref, bg_ref, g_ref, acc_ref):
    i = pl.program_id(0)
    cnt = jnp.maximum(ca_ref[:, 0:1] + cb_ref[:, 0:1], 1.0)
    s = jnp.concatenate([slo_ref[...], shi_ref[...]], axis=1) / cnt
    acc = jnp.dot(prev_ref[...], wn_ref[...], preferred_element_type=jnp.float32)
    acc += jnp.dot(s, wi_ref[...], preferred_element_type=jnp.float32)
    n3 = jnp.maximum(acc + b_ref[...], 0.0)

    @pl.when(i == 0)
    def _():
        acc_ref[...] = jnp.zeros_like(acc_ref)

    acc_ref[...] += jnp.sum(n3.reshape(_BN // 8, 8, 64), axis=0)

    nmean = jnp.sum(acc_ref[...], axis=0, keepdims=True) * (1.0 / _N)
    emean = jnp.sum(esum_ref[...], axis=0, keepdims=True) * (1.0 / _E)
    g = jnp.dot(nmean, wgn_ref[...], preferred_element_type=jnp.float32)
    g += jnp.dot(emean, wge_ref[...], preferred_element_type=jnp.float32)
    g_ref[...] = g + bg_ref[...]


def _tc_node3(n2, s_lo, s_hi, cnt_a, cnt_b, e3sum, Wn3_n, Wn3_i, bn3,
              Wg_n, Wg_e, bg):
    grid = _N // _BN
    return pl.pallas_call(
        _node3_body,
        grid=(grid,),
        in_specs=[
            pl.BlockSpec((_BN, 128), lambda i: (i, 0)),
            pl.BlockSpec((_BN, 32), lambda i: (i, 0)),
            pl.BlockSpec((_BN, 32), lambda i: (i, 0)),
            pl.BlockSpec((_BN, 16), lambda i: (i, 0)),
            pl.BlockSpec((_BN, 16), lambda i: (i, 0)),
            pl.BlockSpec((8, 64), lambda i: (0, 0)),
            pl.BlockSpec((128, 64), lambda i: (0, 0)),
            pl.BlockSpec((64, 64), lambda i: (0, 0)),
            pl.BlockSpec((1, 64), lambda i: (0, 0)),
            pl.BlockSpec((64, 128), lambda i: (0, 0)),
            pl.BlockSpec((64, 128), lambda i: (0, 0)),
            pl.BlockSpec((1, 128), lambda i: (0, 0)),
        ],
        out_specs=pl.BlockSpec((1, 128), lambda i: (0, 0)),
        out_shape=jax.ShapeDtypeStruct((1, 128), jnp.float32),
        scratch_shapes=[pltpu.VMEM((8, 64), jnp.float32)],
    )(n2, s_lo, s_hi, cnt_a, cnt_b, e3sum, Wn3_n, Wn3_i, bn3.reshape(1, 64),
      Wg_n, Wg_e, bg.reshape(1, 128))


_sc_gather16c = _make_sc_gather(_N, 16, _E, with_counts=True)
_sc_gather128 = _make_sc_gather(_N, 128, _E)
_sc_gather64 = _make_sc_gather(_N, 64, _E)
_sc_segsum128 = _make_sc_segsum(_E, _N, 128)
_sc_segsum64 = _make_sc_segsum(_E, _N, 64)
_sc_segsum32 = _make_sc_segsum(_E, _N, 32)


def kernel(x, edge_attr, edge_index, We1_e, We1_s, be1, Wn1_n, Wn1_i, bn1,
           We2_e, We2_s, be2, Wn2_n, Wn2_i, bn2, We3_e, We3_s, be3,
           Wn3_n, Wn3_i, bn3, Wg_n, Wg_e, bg):
    senders = edge_index[0].reshape(NW, _E // NW // 80, 80)
    recv_nw = edge_index[1].reshape(NW, _E // NW // 80, 80)
    recv_ns = edge_index[1].reshape(NS, _E // NS // 80, 80)
    x_pad = jnp.pad(x, ((0, 0), (0, 2)))
    We1_s_pad = jnp.pad(We1_s, ((0, 2), (0, 0)))

    xs, cnt_a, cnt_b = _sc_gather16c(x_pad, senders, recv_nw)
    e1_lo, e1_hi = _tc_edge1(edge_attr, xs, We1_e, We1_s_pad, be1)
    s1_lo, s1_hi = _sc_segsum128(e1_lo, e1_hi, recv_ns)
    n1, m1 = _tc_node(x, s1_lo, s1_hi, cnt_a, cnt_b, Wn1_n, Wn1_i, bn1, We2_s)

    g2 = _sc_gather128(m1, senders)
    e2_lo, e2_hi = _tc_edge2(e1_lo, e1_hi, g2, We2_e, be2)
    s2_lo, s2_hi = _sc_segsum64(e2_lo, e2_hi, recv_ns)
    n2, m2 = _tc_node(n1, s2_lo, s2_hi, cnt_a, cnt_b, Wn2_n, Wn2_i, bn2,
                      We3_s)

    g3 = _sc_gather64(m2, senders)
    e3_lo, e3_hi, e3sum = _tc_edge3(e2_lo, e2_hi, g3, We3_e, be3)
    s3_lo, s3_hi = _sc_segsum32(e3_lo, e3_hi, recv_ns)
    g = _tc_node3(n2, s3_lo, s3_hi, cnt_a, cnt_b, e3sum, Wn3_n, Wn3_i, bn3,
                  Wg_n, Wg_e, bg)
    return g.reshape(128)


# speedup vs baseline: 2.7091x; 1.3581x over previous
import functools

import jax
import jax.numpy as jnp
from jax import lax
from jax.experimental import pallas as pl
from jax.experimental.pallas import tpu as pltpu
from jax.experimental.pallas import tpu_sc as plsc

NC = 2
NS = 16
NW = NC * NS

_N = 10000
_E = 320000


def _make_sc_gather(V, D, E, with_counts=False):
    ew = E // NW
    C = 80
    iters = ew // C
    rows_pt = _N // NS
    ZR = 25
    assert ew % C == 0 and C % 8 == 0

    mesh = plsc.VectorSubcoreMesh(core_axis_name="c", subcore_axis_name="s")

    out_type = [jax.ShapeDtypeStruct((E, D), jnp.float32)]
    scratch = [
        pltpu.VMEM((iters, C), jnp.int32),
        pltpu.VMEM((C, D), jnp.float32),
        pltpu.VMEM((C, D), jnp.float32),
        pltpu.SemaphoreType.DMA,
        pltpu.SemaphoreType.DMA,
        pltpu.SemaphoreType.DMA,
        pltpu.SemaphoreType.DMA,
    ]
    if with_counts:
        out_type += [jax.ShapeDtypeStruct((_N, 16), jnp.float32),
                     jax.ShapeDtypeStruct((_N, 16), jnp.float32)]
        scratch += [
            pltpu.VMEM((iters, C), jnp.int32),
            pltpu.VMEM_SHARED((_N, 16), jnp.float32),
            pltpu.VMEM((C, 16), jnp.float32),
            pltpu.SemaphoreType.DMA,
        ]

    def body(*refs):
        if with_counts:
            (table_hbm, idx_hbm, ridx_hbm, out_hbm, cnt_a, cnt_b,
             idx_all, rows0, rows1, g0, g1, w0, w1,
             ridx_all, cacc, ones_v, csem) = refs
        else:
            (table_hbm, idx_hbm, out_hbm,
             idx_all, rows0, rows1, g0, g1, w0, w1) = refs
        cid = lax.axis_index("c")
        tid = lax.axis_index("s")
        wid = tid * NC + cid
        base = wid * ew
        pltpu.sync_copy(idx_hbm.at[wid], idx_all)

        if with_counts:
            pltpu.sync_copy(ridx_hbm.at[wid], ridx_all)
            zero16 = jnp.zeros((16,), jnp.float32)
            one16 = jnp.ones((16,), jnp.float32)
            for r in range(ZR):
                rows0[r, pl.ds(0, 16)] = zero16
            for r in range(C):
                ones_v[r, pl.ds(0, 16)] = one16

            def zinit(j, _):
                r0 = tid * rows_pt + j * ZR
                pltpu.sync_copy(rows0.at[pl.ds(0, ZR), pl.ds(0, 16)],
                                cacc.at[pl.ds(r0, ZR), :])
                return 0

            lax.fori_loop(0, rows_pt // ZR, zinit, 0)
            plsc.subcore_barrier()

        pltpu.async_copy(table_hbm.at[idx_all.at[0]], rows0, g0)

        def phase(j, cur, nxt, gcur, gnxt, wcur, wnxt):
            pltpu.make_async_copy(table_hbm.at[idx_all.at[j]], cur, gcur).wait()
            pltpu.async_copy(cur, out_hbm.at[pl.ds(base + j * C, C), :], wcur)
            if with_counts:
                @pl.when(j >= 1)
                def _():
                    pltpu.make_async_copy(
                        ones_v, cacc.at[ridx_all.at[0]], csem).wait()
                pltpu.async_copy(ones_v, cacc.at[ridx_all.at[j]], csem,
                                 add=True)

            @pl.when(j + 1 < iters)
            def _():
                @pl.when(j >= 1)
                def _():
                    pltpu.make_async_copy(
                        nxt, out_hbm.at[pl.ds(base, C), :], wnxt).wait()
                pltpu.async_copy(table_hbm.at[idx_all.at[j + 1]], nxt, gnxt)

        def loop_body(j, _):
            @pl.when(j % 2 == 0)
            def _():
                phase(j, rows0, rows1, g0, g1, w0, w1)

            @pl.when(j % 2 == 1)
            def _():
                phase(j, rows1, rows0, g1, g0, w1, w0)
            return 0

        lax.fori_loop(0, iters, loop_body, 0)
        pltpu.make_async_copy(rows0, out_hbm.at[pl.ds(base, C), :], w0).wait()
        pltpu.make_async_copy(rows1, out_hbm.at[pl.ds(base, C), :], w1).wait()

        if with_counts:
            pltpu.make_async_copy(ones_v, cacc.at[ridx_all.at[0]], csem).wait()
            plsc.subcore_barrier()

            def drain(j, _):
                r0 = tid * rows_pt + j * ZR
                stg = rows0.at[pl.ds(0, ZR), pl.ds(0, 16)]
                pltpu.sync_copy(cacc.at[pl.ds(r0, ZR), :], stg)

                @pl.when(cid == 0)
                def _():
                    pltpu.sync_copy(stg, cnt_a.at[pl.ds(r0, ZR), :])

                @pl.when(cid == 1)
                def _():
                    pltpu.sync_copy(stg, cnt_b.at[pl.ds(r0, ZR), :])
                return 0

            lax.fori_loop(0, rows_pt // ZR, drain, 0)

    return functools.partial(
        pl.kernel, mesh=mesh, out_type=tuple(out_type) if with_counts
        else out_type[0],
        scratch_types=scratch,
        compiler_params=pltpu.CompilerParams(use_tc_tiling_on_sc=False),
    )(body)


def _make_sc_segsum(E, N, D2):
    ew = E // NS
    C = 80
    iters = ew // C
    rows_pt = N // NS
    ZR = 25
    assert ew % C == 0 and rows_pt % ZR == 0

    mesh = plsc.VectorSubcoreMesh(core_axis_name="c", subcore_axis_name="s")

    out_type = [
        jax.ShapeDtypeStruct((N, D2), jnp.float32),
        jax.ShapeDtypeStruct((N, D2), jnp.float32),
    ]
    scratch = [
        pltpu.VMEM_SHARED((N, D2), jnp.float32),
        pltpu.VMEM((C, D2), jnp.float32),
        pltpu.VMEM((C, D2), jnp.float32),
        pltpu.VMEM((iters, C), jnp.int32),
        pltpu.SemaphoreType.DMA,
        pltpu.SemaphoreType.DMA,
        pltpu.SemaphoreType.DMA,
        pltpu.SemaphoreType.DMA,
    ]

    def body(e_lo, e_hi, recv, out_lo, out_hi,
             acc, eb0, eb1, idx_all, r0s, r1s, s0s, s1s):
        cid = lax.axis_index("c")
        tid = lax.axis_index("s")
        base = tid * ew

        pltpu.sync_copy(recv.at[tid], idx_all)

        def rstart(j, buf, sem):
            @pl.when(cid == 0)
            def _():
                pltpu.async_copy(e_lo.at[pl.ds(base + j * C, C), :], buf, sem)

            @pl.when(cid == 1)
            def _():
                pltpu.async_copy(e_hi.at[pl.ds(base + j * C, C), :], buf, sem)

        def rwait(buf, sem):
            pltpu.make_async_copy(
                e_lo.at[pl.ds(base, C), :], buf, sem).wait()

        zero16 = jnp.zeros((16,), jnp.float32)
        for r in range(ZR):
            for q in range(D2 // 16):
                eb0[r, pl.ds(q * 16, 16)] = zero16

        def zinit(j, _):
            r0 = tid * rows_pt + j * ZR
            pltpu.sync_copy(eb0.at[pl.ds(0, ZR), :], acc.at[pl.ds(r0, ZR), :])
            return 0

        lax.fori_loop(0, rows_pt // ZR, zinit, 0)
        plsc.subcore_barrier()
        rstart(0, eb0, r0s)

        def swait(buf, sem):
            pltpu.make_async_copy(buf, acc.at[idx_all.at[0]], sem).wait()

        def phase(j, cur, nxt, rcur, rnxt, scur, snxt):
            rwait(cur, rcur)
            pltpu.async_copy(cur, acc.at[idx_all.at[j]], scur, add=True)

            @pl.when(j + 1 < iters)
            def _():
                @pl.when(j >= 1)
                def _():
                    swait(nxt, snxt)
                rstart(j + 1, nxt, rnxt)

        def chunk(j, _):
            @pl.when(j % 2 == 0)
            def _():
                phase(j, eb0, eb1, r0s, r1s, s0s, s1s)

            @pl.when(j % 2 == 1)
            def _():
                phase(j, eb1, eb0, r1s, r0s, s1s, s0s)
            return 0

        lax.fori_loop(0, iters, chunk, 0)
        swait(eb0, s0s)
        swait(eb1, s1s)
        plsc.subcore_barrier()

        def drain(j, _):
            r0 = tid * rows_pt + j * ZR
            pltpu.sync_copy(acc.at[pl.ds(r0, ZR), :], eb0.at[pl.ds(0, ZR), :])

            @pl.when(cid == 0)
            def _():
                pltpu.sync_copy(eb0.at[pl.ds(0, ZR), :],
                                out_lo.at[pl.ds(r0, ZR), :])

            @pl.when(cid == 1)
            def _():
                pltpu.sync_copy(eb0.at[pl.ds(0, ZR), :],
                                out_hi.at[pl.ds(r0, ZR), :])
            return 0

        lax.fori_loop(0, rows_pt // ZR, drain, 0)

    return functools.partial(
        pl.kernel, mesh=mesh, out_type=tuple(out_type),
        scratch_types=scratch,
        compiler_params=pltpu.CompilerParams(use_tc_tiling_on_sc=False),
    )(body)


def _make_sc_segsum_esplit(E, N, D):
    ew = E // NW
    C = 80
    iters = ew // C
    rows_pt = N // NS
    ZR = 25
    assert ew % C == 0 and rows_pt % ZR == 0

    mesh = plsc.VectorSubcoreMesh(core_axis_name="c", subcore_axis_name="s")

    out_type = [
        jax.ShapeDtypeStruct((N, D), jnp.float32),
        jax.ShapeDtypeStruct((N, D), jnp.float32),
    ]
    scratch = [
        pltpu.VMEM_SHARED((N, D), jnp.float32),
        pltpu.VMEM((C, D), jnp.float32),
        pltpu.VMEM((C, D), jnp.float32),
        pltpu.VMEM((iters, C), jnp.int32),
        pltpu.SemaphoreType.DMA,
        pltpu.SemaphoreType.DMA,
        pltpu.SemaphoreType.DMA,
        pltpu.SemaphoreType.DMA,
    ]

    def body(e_hbm, recv, out_a, out_b,
             acc, eb0, eb1, idx_all, r0s, r1s, s0s, s1s):
        cid = lax.axis_index("c")
        tid = lax.axis_index("s")
        wid = tid * NC + cid
        base = wid * ew

        pltpu.sync_copy(recv.at[wid], idx_all)

        zero16 = jnp.zeros((16,), jnp.float32)
        for r in range(ZR):
            for q in range(D // 16):
                eb0[r, pl.ds(q * 16, 16)] = zero16

        def zinit(j, _):
            r0 = tid * rows_pt + j * ZR
            pltpu.sync_copy(eb0.at[pl.ds(0, ZR), :], acc.at[pl.ds(r0, ZR), :])
            return 0

        lax.fori_loop(0, rows_pt // ZR, zinit, 0)
        plsc.subcore_barrier()

        def rstart(j, buf, sem):
            pltpu.async_copy(e_hbm.at[pl.ds(base + j * C, C), :], buf, sem)

        def rwait(buf, sem):
            pltpu.make_async_copy(e_hbm.at[pl.ds(base, C), :], buf, sem).wait()

        def swait(buf, sem):
            pltpu.make_async_copy(buf, acc.at[idx_all.at[0]], sem).wait()

        rstart(0, eb0, r0s)

        def phase(j, cur, nxt, rcur, rnxt, scur, snxt):
            rwait(cur, rcur)
            pltpu.async_copy(cur, acc.at[idx_all.at[j]], scur, add=True)

            @pl.when(j + 1 < iters)
            def _():
                @pl.when(j >= 1)
                def _():
                    swait(nxt, snxt)
                rstart(j + 1, nxt, rnxt)

        def chunk(j, _):
            @pl.when(j % 2 == 0)
            def _():
                phase(j, eb0, eb1, r0s, r1s, s0s, s1s)

            @pl.when(j % 2 == 1)
            def _():
                phase(j, eb1, eb0, r1s, r0s, s1s, s0s)
            return 0

        lax.fori_loop(0, iters, chunk, 0)
        swait(eb0, s0s)
        swait(eb1, s1s)
        plsc.subcore_barrier()

        def drain(j, _):
            r0 = tid * rows_pt + j * ZR
            pltpu.sync_copy(acc.at[pl.ds(r0, ZR), :], eb0.at[pl.ds(0, ZR), :])

            @pl.when(cid == 0)
            def _():
                pltpu.sync_copy(eb0.at[pl.ds(0, ZR), :],
                                out_a.at[pl.ds(r0, ZR), :])

            @pl.when(cid == 1)
            def _():
                pltpu.sync_copy(eb0.at[pl.ds(0, ZR), :],
                                out_b.at[pl.ds(r0, ZR), :])
            return 0

        lax.fori_loop(0, rows_pt // ZR, drain, 0)

    return functools.partial(
        pl.kernel, mesh=mesh, out_type=tuple(out_type),
        scratch_types=scratch,
        compiler_params=pltpu.CompilerParams(use_tc_tiling_on_sc=False),
    )(body)


_BE = 1000


def _edge1_body(ea_ref, xs_ref, we_ref, ws_ref, b_ref, lo_ref, hi_ref):
    acc = jnp.dot(ea_ref[...], we_ref[...], preferred_element_type=jnp.float32)
    acc += jnp.dot(xs_ref[...], ws_ref[...], preferred_element_type=jnp.float32)
    e = jnp.maximum(acc + b_ref[...], 0.0)
    lo_ref[...] = e[:, :128]
    hi_ref[...] = e[:, 128:]


def _tc_edge1(edge_attr, xs, We1_e, We1_s_pad, be1):
    grid = _E // _BE
    return pl.pallas_call(
        _edge1_body,
        grid=(grid,),
        in_specs=[
            pl.BlockSpec((_BE, 10), lambda i: (i, 0)),
            pl.BlockSpec((_BE, 128), lambda i: (i, 0)),
            pl.BlockSpec((10, 256), lambda i: (0, 0)),
            pl.BlockSpec((128, 256), lambda i: (0, 0)),
            pl.BlockSpec((1, 256), lambda i: (0, 0)),
        ],
        out_specs=[
            pl.BlockSpec((_BE, 128), lambda i: (i, 0)),
            pl.BlockSpec((_BE, 128), lambda i: (i, 0)),
        ],
        out_shape=[
            jax.ShapeDtypeStruct((_E, 128), jnp.float32),
            jax.ShapeDtypeStruct((_E, 128), jnp.float32),
        ],
    )(edge_attr, xs, We1_e, We1_s_pad, be1.reshape(1, 256))


def _edge2_body(lo_ref, hi_ref, g_ref, w_ref, b_ref, out_ref):
    acc = jnp.dot(lo_ref[...], w_ref[:128, :], preferred_element_type=jnp.float32)
    acc += jnp.dot(hi_ref[...], w_ref[128:, :], preferred_element_type=jnp.float32)
    out_ref[...] = jnp.maximum(acc + g_ref[...] + b_ref[...], 0.0)


def _tc_edge2(e1_lo, e1_hi, g2, We2_e, be2):
    grid = _E // _BE
    return pl.pallas_call(
        _edge2_body,
        grid=(grid,),
        in_specs=[
            pl.BlockSpec((_BE, 128), lambda i: (i, 0)),
            pl.BlockSpec((_BE, 128), lambda i: (i, 0)),
            pl.BlockSpec((_BE, 128), lambda i: (i, 0)),
            pl.BlockSpec((256, 128), lambda i: (0, 0)),
            pl.BlockSpec((1, 128), lambda i: (0, 0)),
        ],
        out_specs=pl.BlockSpec((_BE, 128), lambda i: (i, 0)),
        out_shape=jax.ShapeDtypeStruct((_E, 128), jnp.float32),
    )(e1_lo, e1_hi, g2, We2_e, be2.reshape(1, 128))


def _edge3_body(e2_ref, g_ref, w_ref, b_ref, out_ref, sum_ref):
    i = pl.program_id(0)
    acc = jnp.dot(e2_ref[...], w_ref[...], preferred_element_type=jnp.float32)
    e = jnp.maximum(acc + g_ref[:, :64] + b_ref[...], 0.0)
    out_ref[...] = jnp.concatenate([e, jnp.zeros_like(e)], axis=1)

    @pl.when(i == 0)
    def _():
        sum_ref[...] = jnp.zeros_like(sum_ref)

    sum_ref[...] += jnp.sum(e.reshape(_BE // 8, 8, 64), axis=0)


def _tc_edge3(e2, g3, We3_e, be3):
    grid = _E // _BE
    return pl.pallas_call(
        _edge3_body,
        grid=(grid,),
        in_specs=[
            pl.BlockSpec((_BE, 128), lambda i: (i, 0)),
            pl.BlockSpec((_BE, 128), lambda i: (i, 0)),
            pl.BlockSpec((128, 64), lambda i: (0, 0)),
            pl.BlockSpec((1, 64), lambda i: (0, 0)),
        ],
        out_specs=[
            pl.BlockSpec((_BE, 128), lambda i: (i, 0)),
            pl.BlockSpec((8, 64), lambda i: (0, 0)),
        ],
        out_shape=[
            jax.ShapeDtypeStruct((_E, 128), jnp.float32),
            jax.ShapeDtypeStruct((8, 64), jnp.float32),
        ],
    )(e2, g3, We3_e, be3.reshape(1, 64))


_BN = 1000


def _node_body(prev_ref, slo_ref, shi_ref, ca_ref, cb_ref, wn_ref, wi_ref,
               b_ref, ws_ref, n_ref, m_ref):
    cnt = jnp.maximum(ca_ref[:, 0:1] + cb_ref[:, 0:1], 1.0)
    s = jnp.concatenate([slo_ref[...], shi_ref[...]], axis=1) / cnt
    acc = jnp.dot(prev_ref[...], wn_ref[...], preferred_element_type=jnp.float32)
    acc += jnp.dot(s, wi_ref[...], preferred_element_type=jnp.float32)
    n = jnp.maximum(acc + b_ref[...], 0.0)
    n_ref[...] = n
    m_ref[...] = jnp.dot(n, ws_ref[...], preferred_element_type=jnp.float32)


def _tc_node(prev, s_lo, s_hi, cnt_a, cnt_b, Wn, Wi, b, Ws):
    d_prev = prev.shape[1]
    d2 = s_lo.shape[1]
    d_out = Wn.shape[1]
    d_m = Ws.shape[1]
    grid = _N // _BN
    return pl.pallas_call(
        _node_body,
        grid=(grid,),
        in_specs=[
            pl.BlockSpec((_BN, d_prev), lambda i: (i, 0)),
            pl.BlockSpec((_BN, d2), lambda i: (i, 0)),
            pl.BlockSpec((_BN, d2), lambda i: (i, 0)),
            pl.BlockSpec((_BN, 16), lambda i: (i, 0)),
            pl.BlockSpec((_BN, 16), lambda i: (i, 0)),
            pl.BlockSpec((d_prev, d_out), lambda i: (0, 0)),
            pl.BlockSpec((2 * d2, d_out), lambda i: (0, 0)),
            pl.BlockSpec((1, d_out), lambda i: (0, 0)),
            pl.BlockSpec((d_out, d_m), lambda i: (0, 0)),
        ],
        out_specs=[
            pl.BlockSpec((_BN, d_out), lambda i: (i, 0)),
            pl.BlockSpec((_BN, d_m), lambda i: (i, 0)),
        ],
        out_shape=[
            jax.ShapeDtypeStruct((_N, d_out), jnp.float32),
            jax.ShapeDtypeStruct((_N, d_m), jnp.float32),
        ],
    )(prev, s_lo, s_hi, cnt_a, cnt_b, Wn, Wi, b.reshape(1, d_out), Ws)


def _node_sum_body(prev_ref, sa_ref, sb_ref, ca_ref, cb_ref, wn_ref, wi_ref,
                   b_ref, ws_ref, n_ref, m_ref):
    cnt = jnp.maximum(ca_ref[:, 0:1] + cb_ref[:, 0:1], 1.0)
    s = (sa_ref[...] + sb_ref[...]) / cnt
    acc = jnp.dot(prev_ref[...], wn_ref[...], preferred_element_type=jnp.float32)
    acc += jnp.dot(s, wi_ref[...], preferred_element_type=jnp.float32)
    n = jnp.maximum(acc + b_ref[...], 0.0)
    n_ref[...] = n
    m_ref[...] = jnp.dot(n, ws_ref[...], preferred_element_type=jnp.float32)


def _tc_node_sum(prev, s_a, s_b, cnt_a, cnt_b, Wn, Wi, b, Ws):
    d_prev = prev.shape[1]
    d_s = s_a.shape[1]
    d_out = Wn.shape[1]
    d_m = Ws.shape[1]
    grid = _N // _BN
    return pl.pallas_call(
        _node_sum_body,
        grid=(grid,),
        in_specs=[
            pl.BlockSpec((_BN, d_prev), lambda i: (i, 0)),
            pl.BlockSpec((_BN, d_s), lambda i: (i, 0)),
            pl.BlockSpec((_BN, d_s), lambda i: (i, 0)),
            pl.BlockSpec((_BN, 16), lambda i: (i, 0)),
            pl.BlockSpec((_BN, 16), lambda i: (i, 0)),
            pl.BlockSpec((d_prev, d_out), lambda i: (0, 0)),
            pl.BlockSpec((d_s, d_out), lambda i: (0, 0)),
            pl.BlockSpec((1, d_out), lambda i: (0, 0)),
            pl.BlockSpec((d_out, d_m), lambda i: (0, 0)),
        ],
        out_specs=[
            pl.BlockSpec((_BN, d_out), lambda i: (i, 0)),
            pl.BlockSpec((_BN, d_m), lambda i: (i, 0)),
        ],
        out_shape=[
            jax.ShapeDtypeStruct((_N, d_out), jnp.float32),
            jax.ShapeDtypeStruct((_N, d_m), jnp.float32),
        ],
    )(prev, s_a, s_b, cnt_a, cnt_b, Wn, Wi, b.reshape(1, d_out), Ws)


def _node3_body(prev_ref, sa_ref, sb_ref, ca_ref, cb_ref, esum_ref, wn_ref,
                wi_ref, b_ref, wgn_ref, wge_ref, bg_ref, g_ref, acc_ref):
    i = pl.program_id(0)
    cnt = jnp.maximum(ca_ref[:, 0:1] + cb_ref[:, 0:1], 1.0)
    s = (sa_ref[...] + sb_ref[...])[:, :64] / cnt
    acc = jnp.dot(prev_ref[...], wn_ref[...], preferred_element_type=jnp.float32)
    acc += jnp.dot(s, wi_ref[...], preferred_element_type=jnp.float32)
    n3 = jnp.maximum(acc + b_ref[...], 0.0)

    @pl.when(i == 0)
    def _():
        acc_ref[...] = jnp.zeros_like(acc_ref)

    acc_ref[...] += jnp.sum(n3.reshape(_BN // 8, 8, 64), axis=0)

    nmean = jnp.sum(acc_ref[...], axis=0, keepdims=True) * (1.0 / _N)
    emean = jnp.sum(esum_ref[...], axis=0, keepdims=True) * (1.0 / _E)
    g = jnp.dot(nmean, wgn_ref[...], preferred_element_type=jnp.float32)
    g += jnp.dot(emean, wge_ref[...], preferred_element_type=jnp.float32)
    g_ref[...] = g + bg_ref[...]


def _tc_node3(n2, s_lo, s_hi, cnt_a, cnt_b, e3sum, Wn3_n, Wn3_i, bn3,
              Wg_n, Wg_e, bg):
    grid = _N // _BN
    return pl.pallas_call(
        _node3_body,
        grid=(grid,),
        in_specs=[
            pl.BlockSpec((_BN, 128), lambda i: (i, 0)),
            pl.BlockSpec((_BN, 128), lambda i: (i, 0)),
            pl.BlockSpec((_BN, 128), lambda i: (i, 0)),
            pl.BlockSpec((_BN, 16), lambda i: (i, 0)),
            pl.BlockSpec((_BN, 16), lambda i: (i, 0)),
            pl.BlockSpec((8, 64), lambda i: (0, 0)),
            pl.BlockSpec((128, 64), lambda i: (0, 0)),
            pl.BlockSpec((64, 64), lambda i: (0, 0)),
            pl.BlockSpec((1, 64), lambda i: (0, 0)),
            pl.BlockSpec((64, 128), lambda i: (0, 0)),
            pl.BlockSpec((64, 128), lambda i: (0, 0)),
            pl.BlockSpec((1, 128), lambda i: (0, 0)),
        ],
        out_specs=pl.BlockSpec((1, 128), lambda i: (0, 0)),
        out_shape=jax.ShapeDtypeStruct((1, 128), jnp.float32),
        scratch_shapes=[pltpu.VMEM((8, 64), jnp.float32)],
    )(n2, s_lo, s_hi, cnt_a, cnt_b, e3sum, Wn3_n, Wn3_i, bn3.reshape(1, 64),
      Wg_n, Wg_e, bg.reshape(1, 128))


_sc_gather128c = _make_sc_gather(_N, 128, _E, with_counts=True)
_sc_gather128 = _make_sc_gather(_N, 128, _E)
_sc_segsum128 = _make_sc_segsum(_E, _N, 128)
_sc_segsum_es = _make_sc_segsum_esplit(_E, _N, 128)


def kernel(x, edge_attr, edge_index, We1_e, We1_s, be1, Wn1_n, Wn1_i, bn1,
           We2_e, We2_s, be2, Wn2_n, Wn2_i, bn2, We3_e, We3_s, be3,
           Wn3_n, Wn3_i, bn3, Wg_n, Wg_e, bg):
    senders = edge_index[0].reshape(NW, _E // NW // 80, 80)
    recv_nw = edge_index[1].reshape(NW, _E // NW // 80, 80)
    recv_ns = edge_index[1].reshape(NS, _E // NS // 80, 80)
    x_pad = jnp.pad(x, ((0, 0), (0, 114)))
    We1_s_pad = jnp.pad(We1_s, ((0, 114), (0, 0)))
    We3_s_pad = jnp.pad(We3_s, ((0, 0), (0, 64)))

    xs, cnt_a, cnt_b = _sc_gather128c(x_pad, senders, recv_nw)
    e1_lo, e1_hi = _tc_edge1(edge_attr, xs, We1_e, We1_s_pad, be1)
    s1_lo, s1_hi = _sc_segsum128(e1_lo, e1_hi, recv_ns)
    n1, m1 = _tc_node(x, s1_lo, s1_hi, cnt_a, cnt_b, Wn1_n, Wn1_i, bn1, We2_s)

    g2 = _sc_gather128(m1, senders)
    e2 = _tc_edge2(e1_lo, e1_hi, g2, We2_e, be2)
    s2a, s2b = _sc_segsum_es(e2, recv_nw)
    n2, m2 = _tc_node_sum(n1, s2a, s2b, cnt_a, cnt_b, Wn2_n, Wn2_i, bn2,
                          We3_s_pad)

    g3 = _sc_gather128(m2, senders)
    e3p, e3sum = _tc_edge3(e2, g3, We3_e, be3)
    s3a, s3b = _sc_segsum_es(e3p, recv_nw)
    g = _tc_node3(n2, s3a, s3b, cnt_a, cnt_b, e3sum, Wn3_n, Wn3_i, bn3,
                  Wg_n, Wg_e, bg)
    return g.reshape(128)


# speedup vs baseline: 3.1447x; 1.1608x over previous
import functools

import jax
import jax.numpy as jnp
from jax import lax
from jax.experimental import pallas as pl
from jax.experimental.pallas import tpu as pltpu
from jax.experimental.pallas import tpu_sc as plsc

NC = 2
NS = 16
NW = NC * NS

_N = 10000
_E = 320000


def _make_sc_gather(V, D, E, with_counts=False):
    ew = E // NW
    C = 80
    iters = ew // C
    rows_pt = _N // NS
    ZR = 25
    assert ew % C == 0 and C % 8 == 0

    mesh = plsc.VectorSubcoreMesh(core_axis_name="c", subcore_axis_name="s")

    out_type = [jax.ShapeDtypeStruct((E, D), jnp.float32)]
    scratch = [
        pltpu.VMEM((iters, C), jnp.int32),
        pltpu.VMEM((C, D), jnp.float32),
        pltpu.VMEM((C, D), jnp.float32),
        pltpu.SemaphoreType.DMA,
        pltpu.SemaphoreType.DMA,
        pltpu.SemaphoreType.DMA,
        pltpu.SemaphoreType.DMA,
    ]
    if with_counts:
        out_type += [jax.ShapeDtypeStruct((_N, 16), jnp.float32),
                     jax.ShapeDtypeStruct((_N, 16), jnp.float32)]
        scratch += [
            pltpu.VMEM((iters, C), jnp.int32),
            pltpu.VMEM_SHARED((_N, 16), jnp.float32),
            pltpu.VMEM((C, 16), jnp.float32),
            pltpu.SemaphoreType.DMA,
        ]

    def body(*refs):
        if with_counts:
            (table_hbm, idx_hbm, ridx_hbm, out_hbm, cnt_a, cnt_b,
             idx_all, rows0, rows1, g0, g1, w0, w1,
             ridx_all, cacc, ones_v, csem) = refs
        else:
            (table_hbm, idx_hbm, out_hbm,
             idx_all, rows0, rows1, g0, g1, w0, w1) = refs
        cid = lax.axis_index("c")
        tid = lax.axis_index("s")
        wid = tid * NC + cid
        base = wid * ew
        pltpu.sync_copy(idx_hbm.at[wid], idx_all)

        if with_counts:
            pltpu.sync_copy(ridx_hbm.at[wid], ridx_all)
            zero16 = jnp.zeros((16,), jnp.float32)
            one16 = jnp.ones((16,), jnp.float32)
            for r in range(ZR):
                rows0[r, pl.ds(0, 16)] = zero16
            for r in range(C):
                ones_v[r, pl.ds(0, 16)] = one16

            def zinit(j, _):
                r0 = tid * rows_pt + j * ZR
                pltpu.sync_copy(rows0.at[pl.ds(0, ZR), pl.ds(0, 16)],
                                cacc.at[pl.ds(r0, ZR), :])
                return 0

            lax.fori_loop(0, rows_pt // ZR, zinit, 0)
            plsc.subcore_barrier()

        pltpu.async_copy(table_hbm.at[idx_all.at[0]], rows0, g0)

        def phase(j, cur, nxt, gcur, gnxt, wcur, wnxt):
            pltpu.make_async_copy(table_hbm.at[idx_all.at[j]], cur, gcur).wait()
            pltpu.async_copy(cur, out_hbm.at[pl.ds(base + j * C, C), :], wcur)
            if with_counts:
                @pl.when(j >= 1)
                def _():
                    pltpu.make_async_copy(
                        ones_v, cacc.at[ridx_all.at[0]], csem).wait()
                pltpu.async_copy(ones_v, cacc.at[ridx_all.at[j]], csem,
                                 add=True)

            @pl.when(j + 1 < iters)
            def _():
                @pl.when(j >= 1)
                def _():
                    pltpu.make_async_copy(
                        nxt, out_hbm.at[pl.ds(base, C), :], wnxt).wait()
                pltpu.async_copy(table_hbm.at[idx_all.at[j + 1]], nxt, gnxt)

        def loop_body(j, _):
            @pl.when(j % 2 == 0)
            def _():
                phase(j, rows0, rows1, g0, g1, w0, w1)

            @pl.when(j % 2 == 1)
            def _():
                phase(j, rows1, rows0, g1, g0, w1, w0)
            return 0

        lax.fori_loop(0, iters, loop_body, 0)
        pltpu.make_async_copy(rows0, out_hbm.at[pl.ds(base, C), :], w0).wait()
        pltpu.make_async_copy(rows1, out_hbm.at[pl.ds(base, C), :], w1).wait()

        if with_counts:
            pltpu.make_async_copy(ones_v, cacc.at[ridx_all.at[0]], csem).wait()
            plsc.subcore_barrier()

            def drain(j, _):
                r0 = tid * rows_pt + j * ZR
                stg = rows0.at[pl.ds(0, ZR), pl.ds(0, 16)]
                pltpu.sync_copy(cacc.at[pl.ds(r0, ZR), :], stg)

                @pl.when(cid == 0)
                def _():
                    pltpu.sync_copy(stg, cnt_a.at[pl.ds(r0, ZR), :])

                @pl.when(cid == 1)
                def _():
                    pltpu.sync_copy(stg, cnt_b.at[pl.ds(r0, ZR), :])
                return 0

            lax.fori_loop(0, rows_pt // ZR, drain, 0)

    return functools.partial(
        pl.kernel, mesh=mesh, out_type=tuple(out_type) if with_counts
        else out_type[0],
        scratch_types=scratch,
        compiler_params=pltpu.CompilerParams(use_tc_tiling_on_sc=False),
    )(body)


def _make_sc_segsum(E, N, D2):
    ew = E // NS
    C = 80
    iters = ew // C
    rows_pt = N // NS
    ZR = 25
    assert ew % C == 0 and rows_pt % ZR == 0

    mesh = plsc.VectorSubcoreMesh(core_axis_name="c", subcore_axis_name="s")

    out_type = [
        jax.ShapeDtypeStruct((N, D2), jnp.float32),
        jax.ShapeDtypeStruct((N, D2), jnp.float32),
    ]
    scratch = [
        pltpu.VMEM_SHARED((N, D2), jnp.float32),
        pltpu.VMEM((C, D2), jnp.float32),
        pltpu.VMEM((C, D2), jnp.float32),
        pltpu.VMEM((iters, C), jnp.int32),
        pltpu.SemaphoreType.DMA,
        pltpu.SemaphoreType.DMA,
        pltpu.SemaphoreType.DMA,
        pltpu.SemaphoreType.DMA,
    ]

    def body(e_lo, e_hi, recv, out_lo, out_hi,
             acc, eb0, eb1, idx_all, r0s, r1s, s0s, s1s):
        cid = lax.axis_index("c")
        tid = lax.axis_index("s")
        base = tid * ew

        pltpu.sync_copy(recv.at[tid], idx_all)

        def rstart(j, buf, sem):
            @pl.when(cid == 0)
            def _():
                pltpu.async_copy(e_lo.at[pl.ds(base + j * C, C), :], buf, sem)

            @pl.when(cid == 1)
            def _():
                pltpu.async_copy(e_hi.at[pl.ds(base + j * C, C), :], buf, sem)

        def rwait(buf, sem):
            pltpu.make_async_copy(
                e_lo.at[pl.ds(base, C), :], buf, sem).wait()

        zero16 = jnp.zeros((16,), jnp.float32)
        for r in range(ZR):
            for q in range(D2 // 16):
                eb0[r, pl.ds(q * 16, 16)] = zero16

        def zinit(j, _):
            r0 = tid * rows_pt + j * ZR
            pltpu.sync_copy(eb0.at[pl.ds(0, ZR), :], acc.at[pl.ds(r0, ZR), :])
            return 0

        lax.fori_loop(0, rows_pt // ZR, zinit, 0)
        plsc.subcore_barrier()
        rstart(0, eb0, r0s)

        def swait(buf, sem):
            pltpu.make_async_copy(buf, acc.at[idx_all.at[0]], sem).wait()

        def phase(j, cur, nxt, rcur, rnxt, scur, snxt):
            rwait(cur, rcur)
            pltpu.async_copy(cur, acc.at[idx_all.at[j]], scur, add=True)

            @pl.when(j + 1 < iters)
            def _():
                @pl.when(j >= 1)
                def _():
                    swait(nxt, snxt)
                rstart(j + 1, nxt, rnxt)

        def chunk(j, _):
            @pl.when(j % 2 == 0)
            def _():
                phase(j, eb0, eb1, r0s, r1s, s0s, s1s)

            @pl.when(j % 2 == 1)
            def _():
                phase(j, eb1, eb0, r1s, r0s, s1s, s0s)
            return 0

        lax.fori_loop(0, iters, chunk, 0)
        swait(eb0, s0s)
        swait(eb1, s1s)
        plsc.subcore_barrier()

        def drain(j, _):
            r0 = tid * rows_pt + j * ZR
            pltpu.sync_copy(acc.at[pl.ds(r0, ZR), :], eb0.at[pl.ds(0, ZR), :])

            @pl.when(cid == 0)
            def _():
                pltpu.sync_copy(eb0.at[pl.ds(0, ZR), :],
                                out_lo.at[pl.ds(r0, ZR), :])

            @pl.when(cid == 1)
            def _():
                pltpu.sync_copy(eb0.at[pl.ds(0, ZR), :],
                                out_hi.at[pl.ds(r0, ZR), :])
            return 0

        lax.fori_loop(0, rows_pt // ZR, drain, 0)

    return functools.partial(
        pl.kernel, mesh=mesh, out_type=tuple(out_type),
        scratch_types=scratch,
        compiler_params=pltpu.CompilerParams(use_tc_tiling_on_sc=False),
    )(body)


def _make_sc_segsum_esplit(E, N, D):
    ew = E // NW
    C = 80
    iters = ew // C
    rows_pt = N // NS
    ZR = 25
    assert ew % C == 0 and rows_pt % ZR == 0

    mesh = plsc.VectorSubcoreMesh(core_axis_name="c", subcore_axis_name="s")

    out_type = [
        jax.ShapeDtypeStruct((N, D), jnp.float32),
        jax.ShapeDtypeStruct((N, D), jnp.float32),
    ]
    scratch = [
        pltpu.VMEM_SHARED((N, D), jnp.float32),
        pltpu.VMEM((C, D), jnp.float32),
        pltpu.VMEM((C, D), jnp.float32),
        pltpu.VMEM((iters, C), jnp.int32),
        pltpu.SemaphoreType.DMA,
        pltpu.SemaphoreType.DMA,
        pltpu.SemaphoreType.DMA,
        pltpu.SemaphoreType.DMA,
    ]

    def body(e_hbm, recv, out_a, out_b,
             acc, eb0, eb1, idx_all, r0s, r1s, s0s, s1s):
        cid = lax.axis_index("c")
        tid = lax.axis_index("s")
        wid = tid * NC + cid
        base = wid * ew

        pltpu.sync_copy(recv.at[wid], idx_all)

        zero16 = jnp.zeros((16,), jnp.float32)
        for r in range(ZR):
            for q in range(D // 16):
                eb0[r, pl.ds(q * 16, 16)] = zero16

        def zinit(j, _):
            r0 = tid * rows_pt + j * ZR
            pltpu.sync_copy(eb0.at[pl.ds(0, ZR), :], acc.at[pl.ds(r0, ZR), :])
            return 0

        lax.fori_loop(0, rows_pt // ZR, zinit, 0)
        plsc.subcore_barrier()

        def rstart(j, buf, sem):
            pltpu.async_copy(e_hbm.at[pl.ds(base + j * C, C), :], buf, sem)

        def rwait(buf, sem):
            pltpu.make_async_copy(e_hbm.at[pl.ds(base, C), :], buf, sem).wait()

        def swait(buf, sem):
            pltpu.make_async_copy(buf, acc.at[idx_all.at[0]], sem).wait()

        rstart(0, eb0, r0s)

        def phase(j, cur, nxt, rcur, rnxt, scur, snxt):
            rwait(cur, rcur)
            pltpu.async_copy(cur, acc.at[idx_all.at[j]], scur, add=True)

            @pl.when(j + 1 < iters)
            def _():
                @pl.when(j >= 1)
                def _():
                    swait(nxt, snxt)
                rstart(j + 1, nxt, rnxt)

        def chunk(j, _):
            @pl.when(j % 2 == 0)
            def _():
                phase(j, eb0, eb1, r0s, r1s, s0s, s1s)

            @pl.when(j % 2 == 1)
            def _():
                phase(j, eb1, eb0, r1s, r0s, s1s, s0s)
            return 0

        lax.fori_loop(0, iters, chunk, 0)
        swait(eb0, s0s)
        swait(eb1, s1s)
        plsc.subcore_barrier()

        def drain(j, _):
            r0 = tid * rows_pt + j * ZR
            pltpu.sync_copy(acc.at[pl.ds(r0, ZR), :], eb0.at[pl.ds(0, ZR), :])

            @pl.when(cid == 0)
            def _():
                pltpu.sync_copy(eb0.at[pl.ds(0, ZR), :],
                                out_a.at[pl.ds(r0, ZR), :])

            @pl.when(cid == 1)
            def _():
                pltpu.sync_copy(eb0.at[pl.ds(0, ZR), :],
                                out_b.at[pl.ds(r0, ZR), :])
            return 0

        lax.fori_loop(0, rows_pt // ZR, drain, 0)

    return functools.partial(
        pl.kernel, mesh=mesh, out_type=tuple(out_type),
        scratch_types=scratch,
        compiler_params=pltpu.CompilerParams(use_tc_tiling_on_sc=False),
    )(body)


_BE = 2000


def _edge1_body(ea_ref, xs_ref, we_ref, ws_ref, b_ref, lo_ref, hi_ref):
    acc = jnp.dot(ea_ref[...], we_ref[...], preferred_element_type=jnp.float32)
    acc += jnp.dot(xs_ref[...], ws_ref[...], preferred_element_type=jnp.float32)
    e = jnp.maximum(acc + b_ref[...], 0.0)
    lo_ref[...] = e[:, :128]
    hi_ref[...] = e[:, 128:]


def _tc_edge1(edge_attr, xs, We1_e, We1_s_pad, be1):
    grid = _E // _BE
    return pl.pallas_call(
        _edge1_body,
        grid=(grid,),
        in_specs=[
            pl.BlockSpec((_BE, 10), lambda i: (i, 0)),
            pl.BlockSpec((_BE, 128), lambda i: (i, 0)),
            pl.BlockSpec((10, 256), lambda i: (0, 0)),
            pl.BlockSpec((128, 256), lambda i: (0, 0)),
            pl.BlockSpec((1, 256), lambda i: (0, 0)),
        ],
        out_specs=[
            pl.BlockSpec((_BE, 128), lambda i: (i, 0)),
            pl.BlockSpec((_BE, 128), lambda i: (i, 0)),
        ],
        out_shape=[
            jax.ShapeDtypeStruct((_E, 128), jnp.float32),
            jax.ShapeDtypeStruct((_E, 128), jnp.float32),
        ],
    )(edge_attr, xs, We1_e, We1_s_pad, be1.reshape(1, 256))


def _edge2_body(lo_ref, hi_ref, g_ref, w_ref, b_ref, out_ref):
    acc = jnp.dot(lo_ref[...], w_ref[:128, :], preferred_element_type=jnp.float32)
    acc += jnp.dot(hi_ref[...], w_ref[128:, :], preferred_element_type=jnp.float32)
    out_ref[...] = jnp.maximum(acc + g_ref[...] + b_ref[...], 0.0)


def _tc_edge2(e1_lo, e1_hi, g2, We2_e, be2):
    grid = _E // _BE
    return pl.pallas_call(
        _edge2_body,
        grid=(grid,),
        in_specs=[
            pl.BlockSpec((_BE, 128), lambda i: (i, 0)),
            pl.BlockSpec((_BE, 128), lambda i: (i, 0)),
            pl.BlockSpec((_BE, 128), lambda i: (i, 0)),
            pl.BlockSpec((256, 128), lambda i: (0, 0)),
            pl.BlockSpec((1, 128), lambda i: (0, 0)),
        ],
        out_specs=pl.BlockSpec((_BE, 128), lambda i: (i, 0)),
        out_shape=jax.ShapeDtypeStruct((_E, 128), jnp.float32),
    )(e1_lo, e1_hi, g2, We2_e, be2.reshape(1, 128))


def _edge3_body(e2_ref, g_ref, w_ref, b_ref, out_ref, sum_ref):
    i = pl.program_id(0)
    acc = jnp.dot(e2_ref[...], w_ref[...], preferred_element_type=jnp.float32)
    e = jnp.maximum(acc + g_ref[:, :64] + b_ref[...], 0.0)
    out_ref[...] = jnp.concatenate([e, jnp.zeros_like(e)], axis=1)

    @pl.when(i == 0)
    def _():
        sum_ref[...] = jnp.zeros_like(sum_ref)

    sum_ref[...] += jnp.sum(e.reshape(_BE // 8, 8, 64), axis=0)


def _tc_edge3(e2, g3, We3_e, be3):
    grid = _E // _BE
    return pl.pallas_call(
        _edge3_body,
        grid=(grid,),
        in_specs=[
            pl.BlockSpec((_BE, 128), lambda i: (i, 0)),
            pl.BlockSpec((_BE, 128), lambda i: (i, 0)),
            pl.BlockSpec((128, 64), lambda i: (0, 0)),
            pl.BlockSpec((1, 64), lambda i: (0, 0)),
        ],
        out_specs=[
            pl.BlockSpec((_BE, 128), lambda i: (i, 0)),
            pl.BlockSpec((8, 64), lambda i: (0, 0)),
        ],
        out_shape=[
            jax.ShapeDtypeStruct((_E, 128), jnp.float32),
            jax.ShapeDtypeStruct((8, 64), jnp.float32),
        ],
    )(e2, g3, We3_e, be3.reshape(1, 64))


_BN = 1000


def _node_body(prev_ref, slo_ref, shi_ref, ca_ref, cb_ref, wn_ref, wi_ref,
               b_ref, ws_ref, n_ref, m_ref):
    cnt = jnp.maximum(ca_ref[:, 0:1] + cb_ref[:, 0:1], 1.0)
    s = jnp.concatenate([slo_ref[...], shi_ref[...]], axis=1) / cnt
    acc = jnp.dot(prev_ref[...], wn_ref[...], preferred_element_type=jnp.float32)
    acc += jnp.dot(s, wi_ref[...], preferred_element_type=jnp.float32)
    n = jnp.maximum(acc + b_ref[...], 0.0)
    n_ref[...] = n
    m_ref[...] = jnp.dot(n, ws_ref[...], preferred_element_type=jnp.float32)


def _tc_node(prev, s_lo, s_hi, cnt_a, cnt_b, Wn, Wi, b, Ws):
    d_prev = prev.shape[1]
    d2 = s_lo.shape[1]
    d_out = Wn.shape[1]
    d_m = Ws.shape[1]
    grid = _N // _BN
    return pl.pallas_call(
        _node_body,
        grid=(grid,),
        in_specs=[
            pl.BlockSpec((_BN, d_prev), lambda i: (i, 0)),
            pl.BlockSpec((_BN, d2), lambda i: (i, 0)),
            pl.BlockSpec((_BN, d2), lambda i: (i, 0)),
            pl.BlockSpec((_BN, 16), lambda i: (i, 0)),
            pl.BlockSpec((_BN, 16), lambda i: (i, 0)),
            pl.BlockSpec((d_prev, d_out), lambda i: (0, 0)),
            pl.BlockSpec((2 * d2, d_out), lambda i: (0, 0)),
            pl.BlockSpec((1, d_out), lambda i: (0, 0)),
            pl.BlockSpec((d_out, d_m), lambda i: (0, 0)),
        ],
        out_specs=[
            pl.BlockSpec((_BN, d_out), lambda i: (i, 0)),
            pl.BlockSpec((_BN, d_m), lambda i: (i, 0)),
        ],
        out_shape=[
            jax.ShapeDtypeStruct((_N, d_out), jnp.float32),
            jax.ShapeDtypeStruct((_N, d_m), jnp.float32),
        ],
    )(prev, s_lo, s_hi, cnt_a, cnt_b, Wn, Wi, b.reshape(1, d_out), Ws)


def _node_sum_body(prev_ref, sa_ref, sb_ref, ca_ref, cb_ref, wn_ref, wi_ref,
                   b_ref, ws_ref, n_ref, m_ref):
    cnt = jnp.maximum(ca_ref[:, 0:1] + cb_ref[:, 0:1], 1.0)
    s = (sa_ref[...] + sb_ref[...]) / cnt
    acc = jnp.dot(prev_ref[...], wn_ref[...], preferred_element_type=jnp.float32)
    acc += jnp.dot(s, wi_ref[...], preferred_element_type=jnp.float32)
    n = jnp.maximum(acc + b_ref[...], 0.0)
    n_ref[...] = n
    m_ref[...] = jnp.dot(n, ws_ref[...], preferred_element_type=jnp.float32)


def _tc_node_sum(prev, s_a, s_b, cnt_a, cnt_b, Wn, Wi, b, Ws):
    d_prev = prev.shape[1]
    d_s = s_a.shape[1]
    d_out = Wn.shape[1]
    d_m = Ws.shape[1]
    grid = _N // _BN
    return pl.pallas_call(
        _node_sum_body,
        grid=(grid,),
        in_specs=[
            pl.BlockSpec((_BN, d_prev), lambda i: (i, 0)),
            pl.BlockSpec((_BN, d_s), lambda i: (i, 0)),
            pl.BlockSpec((_BN, d_s), lambda i: (i, 0)),
            pl.BlockSpec((_BN, 16), lambda i: (i, 0)),
            pl.BlockSpec((_BN, 16), lambda i: (i, 0)),
            pl.BlockSpec((d_prev, d_out), lambda i: (0, 0)),
            pl.BlockSpec((d_s, d_out), lambda i: (0, 0)),
            pl.BlockSpec((1, d_out), lambda i: (0, 0)),
            pl.BlockSpec((d_out, d_m), lambda i: (0, 0)),
        ],
        out_specs=[
            pl.BlockSpec((_BN, d_out), lambda i: (i, 0)),
            pl.BlockSpec((_BN, d_m), lambda i: (i, 0)),
        ],
        out_shape=[
            jax.ShapeDtypeStruct((_N, d_out), jnp.float32),
            jax.ShapeDtypeStruct((_N, d_m), jnp.float32),
        ],
    )(prev, s_a, s_b, cnt_a, cnt_b, Wn, Wi, b.reshape(1, d_out), Ws)


def _node3_body(prev_ref, sa_ref, sb_ref, ca_ref, cb_ref, esum_ref, wn_ref,
                wi_ref, b_ref, wgn_ref, wge_ref, bg_ref, g_ref, acc_ref):
    i = pl.program_id(0)
    cnt = jnp.maximum(ca_ref[:, 0:1] + cb_ref[:, 0:1], 1.0)
    s = (sa_ref[...] + sb_ref[...])[:, :64] / cnt
    acc = jnp.dot(prev_ref[...], wn_ref[...], preferred_element_type=jnp.float32)
    acc += jnp.dot(s, wi_ref[...], preferred_element_type=jnp.float32)
    n3 = jnp.maximum(acc + b_ref[...], 0.0)

    @pl.when(i == 0)
    def _():
        acc_ref[...] = jnp.zeros_like(acc_ref)

    acc_ref[...] += jnp.sum(n3.reshape(_BN // 8, 8, 64), axis=0)

    nmean = jnp.sum(acc_ref[...], axis=0, keepdims=True) * (1.0 / _N)
    emean = jnp.sum(esum_ref[...], axis=0, keepdims=True) * (1.0 / _E)
    g = jnp.dot(nmean, wgn_ref[...], preferred_element_type=jnp.float32)
    g += jnp.dot(emean, wge_ref[...], preferred_element_type=jnp.float32)
    g_ref[...] = g + bg_ref[...]


def _tc_node3(n2, s_lo, s_hi, cnt_a, cnt_b, e3sum, Wn3_n, Wn3_i, bn3,
              Wg_n, Wg_e, bg):
    grid = _N // _BN
    return pl.pallas_call(
        _node3_body,
        grid=(grid,),
        in_specs=[
            pl.BlockSpec((_BN, 128), lambda i: (i, 0)),
            pl.BlockSpec((_BN, 128), lambda i: (i, 0)),
            pl.BlockSpec((_BN, 128), lambda i: (i, 0)),
            pl.BlockSpec((_BN, 16), lambda i: (i, 0)),
            pl.BlockSpec((_BN, 16), lambda i: (i, 0)),
            pl.BlockSpec((8, 64), lambda i: (0, 0)),
            pl.BlockSpec((128, 64), lambda i: (0, 0)),
            pl.BlockSpec((64, 64), lambda i: (0, 0)),
            pl.BlockSpec((1, 64), lambda i: (0, 0)),
            pl.BlockSpec((64, 128), lambda i: (0, 0)),
            pl.BlockSpec((64, 128), lambda i: (0, 0)),
            pl.BlockSpec((1, 128), lambda i: (0, 0)),
        ],
        out_specs=pl.BlockSpec((1, 128), lambda i: (0, 0)),
        out_shape=jax.ShapeDtypeStruct((1, 128), jnp.float32),
        scratch_shapes=[pltpu.VMEM((8, 64), jnp.float32)],
    )(n2, s_lo, s_hi, cnt_a, cnt_b, e3sum, Wn3_n, Wn3_i, bn3.reshape(1, 64),
      Wg_n, Wg_e, bg.reshape(1, 128))


_sc_gather128c = _make_sc_gather(_N, 128, _E, with_counts=True)
_sc_gather128 = _make_sc_gather(_N, 128, _E)
_sc_segsum128 = _make_sc_segsum(_E, _N, 128)
_sc_segsum_es = _make_sc_segsum_esplit(_E, _N, 128)


def kernel(x, edge_attr, edge_index, We1_e, We1_s, be1, Wn1_n, Wn1_i, bn1,
           We2_e, We2_s, be2, Wn2_n, Wn2_i, bn2, We3_e, We3_s, be3,
           Wn3_n, Wn3_i, bn3, Wg_n, Wg_e, bg):
    senders = edge_index[0].reshape(NW, _E // NW // 80, 80)
    recv_nw = edge_index[1].reshape(NW, _E // NW // 80, 80)
    recv_ns = edge_index[1].reshape(NS, _E // NS // 80, 80)
    x_pad = jnp.pad(x, ((0, 0), (0, 114)))
    We1_s_pad = jnp.pad(We1_s, ((0, 114), (0, 0)))
    We3_s_pad = jnp.pad(We3_s, ((0, 0), (0, 64)))

    xs, cnt_a, cnt_b = _sc_gather128c(x_pad, senders, recv_nw)
    e1_lo, e1_hi = _tc_edge1(edge_attr, xs, We1_e, We1_s_pad, be1)
    s1_lo, s1_hi = _sc_segsum128(e1_lo, e1_hi, recv_ns)
    n1, m1 = _tc_node(x, s1_lo, s1_hi, cnt_a, cnt_b, Wn1_n, Wn1_i, bn1, We2_s)

    g2 = _sc_gather128(m1, senders)
    e2 = _tc_edge2(e1_lo, e1_hi, g2, We2_e, be2)
    s2a, s2b = _sc_segsum_es(e2, recv_nw)
    n2, m2 = _tc_node_sum(n1, s2a, s2b, cnt_a, cnt_b, Wn2_n, Wn2_i, bn2,
                          We3_s_pad)

    g3 = _sc_gather128(m2, senders)
    e3p, e3sum = _tc_edge3(e2, g3, We3_e, be3)
    s3a, s3b = _sc_segsum_es(e3p, recv_nw)
    g = _tc_node3(n2, s3a, s3b, cnt_a, cnt_b, e3sum, Wn3_n, Wn3_i, bn3,
                  Wg_n, Wg_e, bg)
    return g.reshape(128)


# speedup vs baseline: 3.3617x; 1.0690x over previous
import functools

import jax
import jax.numpy as jnp
from jax import lax
from jax.experimental import pallas as pl
from jax.experimental.pallas import tpu as pltpu
from jax.experimental.pallas import tpu_sc as plsc

NC = 2
NS = 16
NW = NC * NS

_N = 10000
_E = 320000


def _make_sc_gather(V, D, E, with_counts=False):
    ew = E // NW
    C = 80
    iters = ew // C
    rows_pt = _N // NS
    ZR = 25
    assert ew % C == 0 and C % 8 == 0

    mesh = plsc.VectorSubcoreMesh(core_axis_name="c", subcore_axis_name="s")

    out_type = [jax.ShapeDtypeStruct((E, D), jnp.float32)]
    scratch = [
        pltpu.VMEM((iters, C), jnp.int32),
        pltpu.VMEM((C, D), jnp.float32),
        pltpu.VMEM((C, D), jnp.float32),
        pltpu.SemaphoreType.DMA,
        pltpu.SemaphoreType.DMA,
        pltpu.SemaphoreType.DMA,
        pltpu.SemaphoreType.DMA,
    ]
    if with_counts:
        out_type += [jax.ShapeDtypeStruct((_N, 16), jnp.float32),
                     jax.ShapeDtypeStruct((_N, 16), jnp.float32)]
        scratch += [
            pltpu.VMEM((iters, C), jnp.int32),
            pltpu.VMEM_SHARED((_N, 16), jnp.float32),
            pltpu.VMEM((C, 16), jnp.float32),
            pltpu.SemaphoreType.DMA,
        ]

    def body(*refs):
        if with_counts:
            (table_hbm, idx_hbm, ridx_hbm, out_hbm, cnt_a, cnt_b,
             idx_all, rows0, rows1, g0, g1, w0, w1,
             ridx_all, cacc, ones_v, csem) = refs
        else:
            (table_hbm, idx_hbm, out_hbm,
             idx_all, rows0, rows1, g0, g1, w0, w1) = refs
        cid = lax.axis_index("c")
        tid = lax.axis_index("s")
        wid = tid * NC + cid
        base = wid * ew
        pltpu.sync_copy(idx_hbm.at[wid], idx_all)

        if with_counts:
            pltpu.sync_copy(ridx_hbm.at[wid], ridx_all)
            zero16 = jnp.zeros((16,), jnp.float32)
            one16 = jnp.ones((16,), jnp.float32)
            for r in range(ZR):
                rows0[r, pl.ds(0, 16)] = zero16
            for r in range(C):
                ones_v[r, pl.ds(0, 16)] = one16

            def zinit(j, _):
                r0 = tid * rows_pt + j * ZR
                pltpu.sync_copy(rows0.at[pl.ds(0, ZR), pl.ds(0, 16)],
                                cacc.at[pl.ds(r0, ZR), :])
                return 0

            lax.fori_loop(0, rows_pt // ZR, zinit, 0)
            plsc.subcore_barrier()

        pltpu.async_copy(table_hbm.at[idx_all.at[0]], rows0, g0)

        def phase(j, cur, nxt, gcur, gnxt, wcur, wnxt):
            pltpu.make_async_copy(table_hbm.at[idx_all.at[j]], cur, gcur).wait()
            pltpu.async_copy(cur, out_hbm.at[pl.ds(base + j * C, C), :], wcur)
            if with_counts:
                @pl.when(j >= 1)
                def _():
                    pltpu.make_async_copy(
                        ones_v, cacc.at[ridx_all.at[0]], csem).wait()
                pltpu.async_copy(ones_v, cacc.at[ridx_all.at[j]], csem,
                                 add=True)

            @pl.when(j + 1 < iters)
            def _():
                @pl.when(j >= 1)
                def _():
                    pltpu.make_async_copy(
                        nxt, out_hbm.at[pl.ds(base, C), :], wnxt).wait()
                pltpu.async_copy(table_hbm.at[idx_all.at[j + 1]], nxt, gnxt)

        def loop_body(j, _):
            @pl.when(j % 2 == 0)
            def _():
                phase(j, rows0, rows1, g0, g1, w0, w1)

            @pl.when(j % 2 == 1)
            def _():
                phase(j, rows1, rows0, g1, g0, w1, w0)
            return 0

        lax.fori_loop(0, iters, loop_body, 0)
        pltpu.make_async_copy(rows0, out_hbm.at[pl.ds(base, C), :], w0).wait()
        pltpu.make_async_copy(rows1, out_hbm.at[pl.ds(base, C), :], w1).wait()

        if with_counts:
            pltpu.make_async_copy(ones_v, cacc.at[ridx_all.at[0]], csem).wait()
            plsc.subcore_barrier()

            def drain(j, _):
                r0 = tid * rows_pt + j * ZR
                stg = rows0.at[pl.ds(0, ZR), pl.ds(0, 16)]
                pltpu.sync_copy(cacc.at[pl.ds(r0, ZR), :], stg)

                @pl.when(cid == 0)
                def _():
                    pltpu.sync_copy(stg, cnt_a.at[pl.ds(r0, ZR), :])

                @pl.when(cid == 1)
                def _():
                    pltpu.sync_copy(stg, cnt_b.at[pl.ds(r0, ZR), :])
                return 0

            lax.fori_loop(0, rows_pt // ZR, drain, 0)

    return functools.partial(
        pl.kernel, mesh=mesh, out_type=tuple(out_type) if with_counts
        else out_type[0],
        scratch_types=scratch,
        compiler_params=pltpu.CompilerParams(use_tc_tiling_on_sc=False),
    )(body)


def _make_sc_segsum(E, N, D2):
    ew = E // NS
    C = 80
    iters = ew // C
    rows_pt = N // NS
    ZR = 25
    assert ew % C == 0 and rows_pt % ZR == 0

    mesh = plsc.VectorSubcoreMesh(core_axis_name="c", subcore_axis_name="s")

    out_type = [
        jax.ShapeDtypeStruct((N, D2), jnp.float32),
        jax.ShapeDtypeStruct((N, D2), jnp.float32),
    ]
    scratch = [
        pltpu.VMEM_SHARED((N, D2), jnp.float32),
        pltpu.VMEM((C, D2), jnp.float32),
        pltpu.VMEM((C, D2), jnp.float32),
        pltpu.VMEM((iters, C), jnp.int32),
        pltpu.SemaphoreType.DMA,
        pltpu.SemaphoreType.DMA,
        pltpu.SemaphoreType.DMA,
        pltpu.SemaphoreType.DMA,
    ]

    def body(e_lo, e_hi, recv, out_lo, out_hi,
             acc, eb0, eb1, idx_all, r0s, r1s, s0s, s1s):
        cid = lax.axis_index("c")
        tid = lax.axis_index("s")
        base = tid * ew

        pltpu.sync_copy(recv.at[tid], idx_all)

        def rstart(j, buf, sem):
            @pl.when(cid == 0)
            def _():
                pltpu.async_copy(e_lo.at[pl.ds(base + j * C, C), :], buf, sem)

            @pl.when(cid == 1)
            def _():
                pltpu.async_copy(e_hi.at[pl.ds(base + j * C, C), :], buf, sem)

        def rwait(buf, sem):
            pltpu.make_async_copy(
                e_lo.at[pl.ds(base, C), :], buf, sem).wait()

        zero16 = jnp.zeros((16,), jnp.float32)
        for r in range(ZR):
            for q in range(D2 // 16):
                eb0[r, pl.ds(q * 16, 16)] = zero16

        def zinit(j, _):
            r0 = tid * rows_pt + j * ZR
            pltpu.sync_copy(eb0.at[pl.ds(0, ZR), :], acc.at[pl.ds(r0, ZR), :])
            return 0

        lax.fori_loop(0, rows_pt // ZR, zinit, 0)
        plsc.subcore_barrier()
        rstart(0, eb0, r0s)

        def swait(buf, sem):
            pltpu.make_async_copy(buf, acc.at[idx_all.at[0]], sem).wait()

        def phase(j, cur, nxt, rcur, rnxt, scur, snxt):
            rwait(cur, rcur)
            pltpu.async_copy(cur, acc.at[idx_all.at[j]], scur, add=True)

            @pl.when(j + 1 < iters)
            def _():
                @pl.when(j >= 1)
                def _():
                    swait(nxt, snxt)
                rstart(j + 1, nxt, rnxt)

        def chunk(j, _):
            @pl.when(j % 2 == 0)
            def _():
                phase(j, eb0, eb1, r0s, r1s, s0s, s1s)

            @pl.when(j % 2 == 1)
            def _():
                phase(j, eb1, eb0, r1s, r0s, s1s, s0s)
            return 0

        lax.fori_loop(0, iters, chunk, 0)
        swait(eb0, s0s)
        swait(eb1, s1s)
        plsc.subcore_barrier()

        def drain(j, _):
            r0 = tid * rows_pt + j * ZR
            pltpu.sync_copy(acc.at[pl.ds(r0, ZR), :], eb0.at[pl.ds(0, ZR), :])

            @pl.when(cid == 0)
            def _():
                pltpu.sync_copy(eb0.at[pl.ds(0, ZR), :],
                                out_lo.at[pl.ds(r0, ZR), :])

            @pl.when(cid == 1)
            def _():
                pltpu.sync_copy(eb0.at[pl.ds(0, ZR), :],
                                out_hi.at[pl.ds(r0, ZR), :])
            return 0

        lax.fori_loop(0, rows_pt // ZR, drain, 0)

    return functools.partial(
        pl.kernel, mesh=mesh, out_type=tuple(out_type),
        scratch_types=scratch,
        compiler_params=pltpu.CompilerParams(use_tc_tiling_on_sc=False),
    )(body)


def _make_sc_segsum_esplit(E, N, D):
    ew = E // NW
    C = 80
    iters = ew // C
    rows_pt = N // NS
    ZR = 25
    assert ew % C == 0 and rows_pt % ZR == 0

    mesh = plsc.VectorSubcoreMesh(core_axis_name="c", subcore_axis_name="s")

    out_type = [
        jax.ShapeDtypeStruct((N, D), jnp.float32),
        jax.ShapeDtypeStruct((N, D), jnp.float32),
    ]
    scratch = [
        pltpu.VMEM_SHARED((N, D), jnp.float32),
        pltpu.VMEM((C, D), jnp.float32),
        pltpu.VMEM((C, D), jnp.float32),
        pltpu.VMEM((iters, C), jnp.int32),
        pltpu.SemaphoreType.DMA,
        pltpu.SemaphoreType.DMA,
        pltpu.SemaphoreType.DMA,
        pltpu.SemaphoreType.DMA,
    ]

    def body(e_hbm, recv, out_a, out_b,
             acc, eb0, eb1, idx_all, r0s, r1s, s0s, s1s):
        cid = lax.axis_index("c")
        tid = lax.axis_index("s")
        wid = tid * NC + cid
        base = wid * ew

        pltpu.sync_copy(recv.at[wid], idx_all)

        zero16 = jnp.zeros((16,), jnp.float32)
        for r in range(ZR):
            for q in range(D // 16):
                eb0[r, pl.ds(q * 16, 16)] = zero16

        def zinit(j, _):
            r0 = tid * rows_pt + j * ZR
            pltpu.sync_copy(eb0.at[pl.ds(0, ZR), :], acc.at[pl.ds(r0, ZR), :])
            return 0

        lax.fori_loop(0, rows_pt // ZR, zinit, 0)
        plsc.subcore_barrier()

        def rstart(j, buf, sem):
            pltpu.async_copy(e_hbm.at[pl.ds(base + j * C, C), :], buf, sem)

        def rwait(buf, sem):
            pltpu.make_async_copy(e_hbm.at[pl.ds(base, C), :], buf, sem).wait()

        def swait(buf, sem):
            pltpu.make_async_copy(buf, acc.at[idx_all.at[0]], sem).wait()

        rstart(0, eb0, r0s)

        def phase(j, cur, nxt, rcur, rnxt, scur, snxt):
            rwait(cur, rcur)
            pltpu.async_copy(cur, acc.at[idx_all.at[j]], scur, add=True)

            @pl.when(j + 1 < iters)
            def _():
                @pl.when(j >= 1)
                def _():
                    swait(nxt, snxt)
                rstart(j + 1, nxt, rnxt)

        def chunk(j, _):
            @pl.when(j % 2 == 0)
            def _():
                phase(j, eb0, eb1, r0s, r1s, s0s, s1s)

            @pl.when(j % 2 == 1)
            def _():
                phase(j, eb1, eb0, r1s, r0s, s1s, s0s)
            return 0

        lax.fori_loop(0, iters, chunk, 0)
        swait(eb0, s0s)
        swait(eb1, s1s)
        plsc.subcore_barrier()

        def drain(j, _):
            r0 = tid * rows_pt + j * ZR
            pltpu.sync_copy(acc.at[pl.ds(r0, ZR), :], eb0.at[pl.ds(0, ZR), :])

            @pl.when(cid == 0)
            def _():
                pltpu.sync_copy(eb0.at[pl.ds(0, ZR), :],
                                out_a.at[pl.ds(r0, ZR), :])

            @pl.when(cid == 1)
            def _():
                pltpu.sync_copy(eb0.at[pl.ds(0, ZR), :],
                                out_b.at[pl.ds(r0, ZR), :])
            return 0

        lax.fori_loop(0, rows_pt // ZR, drain, 0)

    return functools.partial(
        pl.kernel, mesh=mesh, out_type=tuple(out_type),
        scratch_types=scratch,
        compiler_params=pltpu.CompilerParams(use_tc_tiling_on_sc=False),
    )(body)


_BE = 4000


def _edge1_body(ea_ref, xs_ref, we_ref, ws_ref, b_ref, lo_ref, hi_ref):
    acc = jnp.dot(ea_ref[...], we_ref[...], preferred_element_type=jnp.float32)
    acc += jnp.dot(xs_ref[...], ws_ref[...], preferred_element_type=jnp.float32)
    e = jnp.maximum(acc + b_ref[...], 0.0)
    lo_ref[...] = e[:, :128]
    hi_ref[...] = e[:, 128:]


def _tc_edge1(edge_attr, xs, We1_e, We1_s_pad, be1):
    grid = _E // _BE
    return pl.pallas_call(
        _edge1_body,
        grid=(grid,),
        in_specs=[
            pl.BlockSpec((_BE, 10), lambda i: (i, 0)),
            pl.BlockSpec((_BE, 128), lambda i: (i, 0)),
            pl.BlockSpec((10, 256), lambda i: (0, 0)),
            pl.BlockSpec((128, 256), lambda i: (0, 0)),
            pl.BlockSpec((1, 256), lambda i: (0, 0)),
        ],
        out_specs=[
            pl.BlockSpec((_BE, 128), lambda i: (i, 0)),
            pl.BlockSpec((_BE, 128), lambda i: (i, 0)),
        ],
        out_shape=[
            jax.ShapeDtypeStruct((_E, 128), jnp.float32),
            jax.ShapeDtypeStruct((_E, 128), jnp.float32),
        ],
    )(edge_attr, xs, We1_e, We1_s_pad, be1.reshape(1, 256))


def _edge2_body(lo_ref, hi_ref, g_ref, w_ref, b_ref, out_ref):
    acc = jnp.dot(lo_ref[...], w_ref[:128, :], preferred_element_type=jnp.float32)
    acc += jnp.dot(hi_ref[...], w_ref[128:, :], preferred_element_type=jnp.float32)
    out_ref[...] = jnp.maximum(acc + g_ref[...] + b_ref[...], 0.0)


def _tc_edge2(e1_lo, e1_hi, g2, We2_e, be2):
    grid = _E // _BE
    return pl.pallas_call(
        _edge2_body,
        grid=(grid,),
        in_specs=[
            pl.BlockSpec((_BE, 128), lambda i: (i, 0)),
            pl.BlockSpec((_BE, 128), lambda i: (i, 0)),
            pl.BlockSpec((_BE, 128), lambda i: (i, 0)),
            pl.BlockSpec((256, 128), lambda i: (0, 0)),
            pl.BlockSpec((1, 128), lambda i: (0, 0)),
        ],
        out_specs=pl.BlockSpec((_BE, 128), lambda i: (i, 0)),
        out_shape=jax.ShapeDtypeStruct((_E, 128), jnp.float32),
    )(e1_lo, e1_hi, g2, We2_e, be2.reshape(1, 128))


def _edge3_body(e2_ref, g_ref, w_ref, b_ref, out_ref, sum_ref):
    i = pl.program_id(0)
    acc = jnp.dot(e2_ref[...], w_ref[...], preferred_element_type=jnp.float32)
    e = jnp.maximum(acc + g_ref[:, :64] + b_ref[...], 0.0)
    out_ref[...] = jnp.concatenate([e, jnp.zeros_like(e)], axis=1)

    @pl.when(i == 0)
    def _():
        sum_ref[...] = jnp.zeros_like(sum_ref)

    sum_ref[...] += jnp.sum(e.reshape(_BE // 8, 8, 64), axis=0)


def _tc_edge3(e2, g3, We3_e, be3):
    grid = _E // _BE
    return pl.pallas_call(
        _edge3_body,
        grid=(grid,),
        in_specs=[
            pl.BlockSpec((_BE, 128), lambda i: (i, 0)),
            pl.BlockSpec((_BE, 128), lambda i: (i, 0)),
            pl.BlockSpec((128, 64), lambda i: (0, 0)),
            pl.BlockSpec((1, 64), lambda i: (0, 0)),
        ],
        out_specs=[
            pl.BlockSpec((_BE, 128), lambda i: (i, 0)),
            pl.BlockSpec((8, 64), lambda i: (0, 0)),
        ],
        out_shape=[
            jax.ShapeDtypeStruct((_E, 128), jnp.float32),
            jax.ShapeDtypeStruct((8, 64), jnp.float32),
        ],
    )(e2, g3, We3_e, be3.reshape(1, 64))


_BN = 1000


def _node_body(prev_ref, slo_ref, shi_ref, ca_ref, cb_ref, wn_ref, wi_ref,
               b_ref, ws_ref, n_ref, m_ref):
    cnt = jnp.maximum(ca_ref[:, 0:1] + cb_ref[:, 0:1], 1.0)
    s = jnp.concatenate([slo_ref[...], shi_ref[...]], axis=1) / cnt
    acc = jnp.dot(prev_ref[...], wn_ref[...], preferred_element_type=jnp.float32)
    acc += jnp.dot(s, wi_ref[...], preferred_element_type=jnp.float32)
    n = jnp.maximum(acc + b_ref[...], 0.0)
    n_ref[...] = n
    m_ref[...] = jnp.dot(n, ws_ref[...], preferred_element_type=jnp.float32)


def _tc_node(prev, s_lo, s_hi, cnt_a, cnt_b, Wn, Wi, b, Ws):
    d_prev = prev.shape[1]
    d2 = s_lo.shape[1]
    d_out = Wn.shape[1]
    d_m = Ws.shape[1]
    grid = _N // _BN
    return pl.pallas_call(
        _node_body,
        grid=(grid,),
        in_specs=[
            pl.BlockSpec((_BN, d_prev), lambda i: (i, 0)),
            pl.BlockSpec((_BN, d2), lambda i: (i, 0)),
            pl.BlockSpec((_BN, d2), lambda i: (i, 0)),
            pl.BlockSpec((_BN, 16), lambda i: (i, 0)),
            pl.BlockSpec((_BN, 16), lambda i: (i, 0)),
            pl.BlockSpec((d_prev, d_out), lambda i: (0, 0)),
            pl.BlockSpec((2 * d2, d_out), lambda i: (0, 0)),
            pl.BlockSpec((1, d_out), lambda i: (0, 0)),
            pl.BlockSpec((d_out, d_m), lambda i: (0, 0)),
        ],
        out_specs=[
            pl.BlockSpec((_BN, d_out), lambda i: (i, 0)),
            pl.BlockSpec((_BN, d_m), lambda i: (i, 0)),
        ],
        out_shape=[
            jax.ShapeDtypeStruct((_N, d_out), jnp.float32),
            jax.ShapeDtypeStruct((_N, d_m), jnp.float32),
        ],
    )(prev, s_lo, s_hi, cnt_a, cnt_b, Wn, Wi, b.reshape(1, d_out), Ws)


def _node_sum_body(prev_ref, sa_ref, sb_ref, ca_ref, cb_ref, wn_ref, wi_ref,
                   b_ref, ws_ref, n_ref, m_ref):
    cnt = jnp.maximum(ca_ref[:, 0:1] + cb_ref[:, 0:1], 1.0)
    s = (sa_ref[...] + sb_ref[...]) / cnt
    acc = jnp.dot(prev_ref[...], wn_ref[...], preferred_element_type=jnp.float32)
    acc += jnp.dot(s, wi_ref[...], preferred_element_type=jnp.float32)
    n = jnp.maximum(acc + b_ref[...], 0.0)
    n_ref[...] = n
    m_ref[...] = jnp.dot(n, ws_ref[...], preferred_element_type=jnp.float32)


def _tc_node_sum(prev, s_a, s_b, cnt_a, cnt_b, Wn, Wi, b, Ws):
    d_prev = prev.shape[1]
    d_s = s_a.shape[1]
    d_out = Wn.shape[1]
    d_m = Ws.shape[1]
    grid = _N // _BN
    return pl.pallas_call(
        _node_sum_body,
        grid=(grid,),
        in_specs=[
            pl.BlockSpec((_BN, d_prev), lambda i: (i, 0)),
            pl.BlockSpec((_BN, d_s), lambda i: (i, 0)),
            pl.BlockSpec((_BN, d_s), lambda i: (i, 0)),
            pl.BlockSpec((_BN, 16), lambda i: (i, 0)),
            pl.BlockSpec((_BN, 16), lambda i: (i, 0)),
            pl.BlockSpec((d_prev, d_out), lambda i: (0, 0)),
            pl.BlockSpec((d_s, d_out), lambda i: (0, 0)),
            pl.BlockSpec((1, d_out), lambda i: (0, 0)),
            pl.BlockSpec((d_out, d_m), lambda i: (0, 0)),
        ],
        out_specs=[
            pl.BlockSpec((_BN, d_out), lambda i: (i, 0)),
            pl.BlockSpec((_BN, d_m), lambda i: (i, 0)),
        ],
        out_shape=[
            jax.ShapeDtypeStruct((_N, d_out), jnp.float32),
            jax.ShapeDtypeStruct((_N, d_m), jnp.float32),
        ],
    )(prev, s_a, s_b, cnt_a, cnt_b, Wn, Wi, b.reshape(1, d_out), Ws)


def _node3_body(prev_ref, sa_ref, sb_ref, ca_ref, cb_ref, esum_ref, wn_ref,
                wi_ref, b_ref, wgn_ref, wge_ref, bg_ref, g_ref, acc_ref):
    i = pl.program_id(0)
    cnt = jnp.maximum(ca_ref[:, 0:1] + cb_ref[:, 0:1], 1.0)
    s = (sa_ref[...] + sb_ref[...])[:, :64] / cnt
    acc = jnp.dot(prev_ref[...], wn_ref[...], preferred_element_type=jnp.float32)
    acc += jnp.dot(s, wi_ref[...], preferred_element_type=jnp.float32)
    n3 = jnp.maximum(acc + b_ref[...], 0.0)

    @pl.when(i == 0)
    def _():
        acc_ref[...] = jnp.zeros_like(acc_ref)

    acc_ref[...] += jnp.sum(n3.reshape(_BN // 8, 8, 64), axis=0)

    nmean = jnp.sum(acc_ref[...], axis=0, keepdims=True) * (1.0 / _N)
    emean = jnp.sum(esum_ref[...], axis=0, keepdims=True) * (1.0 / _E)
    g = jnp.dot(nmean, wgn_ref[...], preferred_element_type=jnp.float32)
    g += jnp.dot(emean, wge_ref[...], preferred_element_type=jnp.float32)
    g_ref[...] = g + bg_ref[...]


def _tc_node3(n2, s_lo, s_hi, cnt_a, cnt_b, e3sum, Wn3_n, Wn3_i, bn3,
              Wg_n, Wg_e, bg):
    grid = _N // _BN
    return pl.pallas_call(
        _node3_body,
        grid=(grid,),
        in_specs=[
            pl.BlockSpec((_BN, 128), lambda i: (i, 0)),
            pl.BlockSpec((_BN, 128), lambda i: (i, 0)),
            pl.BlockSpec((_BN, 128), lambda i: (i, 0)),
            pl.BlockSpec((_BN, 16), lambda i: (i, 0)),
            pl.BlockSpec((_BN, 16), lambda i: (i, 0)),
            pl.BlockSpec((8, 64), lambda i: (0, 0)),
            pl.BlockSpec((128, 64), lambda i: (0, 0)),
            pl.BlockSpec((64, 64), lambda i: (0, 0)),
            pl.BlockSpec((1, 64), lambda i: (0, 0)),
            pl.BlockSpec((64, 128), lambda i: (0, 0)),
            pl.BlockSpec((64, 128), lambda i: (0, 0)),
            pl.BlockSpec((1, 128), lambda i: (0, 0)),
        ],
        out_specs=pl.BlockSpec((1, 128), lambda i: (0, 0)),
        out_shape=jax.ShapeDtypeStruct((1, 128), jnp.float32),
        scratch_shapes=[pltpu.VMEM((8, 64), jnp.float32)],
    )(n2, s_lo, s_hi, cnt_a, cnt_b, e3sum, Wn3_n, Wn3_i, bn3.reshape(1, 64),
      Wg_n, Wg_e, bg.reshape(1, 128))


_sc_gather128c = _make_sc_gather(_N, 128, _E, with_counts=True)
_sc_gather128 = _make_sc_gather(_N, 128, _E)
_sc_segsum128 = _make_sc_segsum(_E, _N, 128)
_sc_segsum_es = _make_sc_segsum_esplit(_E, _N, 128)


def kernel(x, edge_attr, edge_index, We1_e, We1_s, be1, Wn1_n, Wn1_i, bn1,
           We2_e, We2_s, be2, Wn2_n, Wn2_i, bn2, We3_e, We3_s, be3,
           Wn3_n, Wn3_i, bn3, Wg_n, Wg_e, bg):
    senders = edge_index[0].reshape(NW, _E // NW // 80, 80)
    recv_nw = edge_index[1].reshape(NW, _E // NW // 80, 80)
    recv_ns = edge_index[1].reshape(NS, _E // NS // 80, 80)
    x_pad = jnp.pad(x, ((0, 0), (0, 114)))
    We1_s_pad = jnp.pad(We1_s, ((0, 114), (0, 0)))
    We3_s_pad = jnp.pad(We3_s, ((0, 0), (0, 64)))

    xs, cnt_a, cnt_b = _sc_gather128c(x_pad, senders, recv_nw)
    e1_lo, e1_hi = _tc_edge1(edge_attr, xs, We1_e, We1_s_pad, be1)
    s1_lo, s1_hi = _sc_segsum128(e1_lo, e1_hi, recv_ns)
    n1, m1 = _tc_node(x, s1_lo, s1_hi, cnt_a, cnt_b, Wn1_n, Wn1_i, bn1, We2_s)

    g2 = _sc_gather128(m1, senders)
    e2 = _tc_edge2(e1_lo, e1_hi, g2, We2_e, be2)
    s2a, s2b = _sc_segsum_es(e2, recv_nw)
    n2, m2 = _tc_node_sum(n1, s2a, s2b, cnt_a, cnt_b, Wn2_n, Wn2_i, bn2,
                          We3_s_pad)

    g3 = _sc_gather128(m2, senders)
    e3p, e3sum = _tc_edge3(e2, g3, We3_e, be3)
    s3a, s3b = _sc_segsum_es(e3p, recv_nw)
    g = _tc_node3(n2, s3a, s3b, cnt_a, cnt_b, e3sum, Wn3_n, Wn3_i, bn3,
                  Wg_n, Wg_e, bg)
    return g.reshape(128)


# speedup vs baseline: 3.3920x; 1.0090x over previous
import functools

import jax
import jax.numpy as jnp
from jax import lax
from jax.experimental import pallas as pl
from jax.experimental.pallas import tpu as pltpu
from jax.experimental.pallas import tpu_sc as plsc

NC = 2
NS = 16
NW = NC * NS

_N = 10000
_E = 320000


def _make_sc_gather(V, D, E, with_counts=False):
    ew = E // NW
    C = 80
    iters = ew // C
    rows_pt = _N // NS
    ZR = 25
    assert ew % C == 0 and C % 8 == 0

    mesh = plsc.VectorSubcoreMesh(core_axis_name="c", subcore_axis_name="s")

    out_type = [jax.ShapeDtypeStruct((E, D), jnp.float32)]
    scratch = [
        pltpu.VMEM((iters, C), jnp.int32),
        pltpu.VMEM((C, D), jnp.float32),
        pltpu.VMEM((C, D), jnp.float32),
        pltpu.SemaphoreType.DMA,
        pltpu.SemaphoreType.DMA,
        pltpu.SemaphoreType.DMA,
        pltpu.SemaphoreType.DMA,
    ]
    if with_counts:
        out_type += [jax.ShapeDtypeStruct((_N, 16), jnp.float32),
                     jax.ShapeDtypeStruct((_N, 16), jnp.float32)]
        scratch += [
            pltpu.VMEM((iters, C), jnp.int32),
            pltpu.VMEM_SHARED((_N, 16), jnp.float32),
            pltpu.VMEM((C, 16), jnp.float32),
            pltpu.SemaphoreType.DMA,
        ]

    def body(*refs):
        if with_counts:
            (table_hbm, idx_hbm, ridx_hbm, out_hbm, cnt_a, cnt_b,
             idx_all, rows0, rows1, g0, g1, w0, w1,
             ridx_all, cacc, ones_v, csem) = refs
        else:
            (table_hbm, idx_hbm, out_hbm,
             idx_all, rows0, rows1, g0, g1, w0, w1) = refs
        cid = lax.axis_index("c")
        tid = lax.axis_index("s")
        wid = tid * NC + cid
        base = wid * ew
        pltpu.sync_copy(idx_hbm.at[wid], idx_all)

        if with_counts:
            pltpu.sync_copy(ridx_hbm.at[wid], ridx_all)
            zero16 = jnp.zeros((16,), jnp.float32)
            one16 = jnp.ones((16,), jnp.float32)
            for r in range(ZR):
                rows0[r, pl.ds(0, 16)] = zero16
            for r in range(C):
                ones_v[r, pl.ds(0, 16)] = one16

            def zinit(j, _):
                r0 = tid * rows_pt + j * ZR
                pltpu.sync_copy(rows0.at[pl.ds(0, ZR), pl.ds(0, 16)],
                                cacc.at[pl.ds(r0, ZR), :])
                return 0

            lax.fori_loop(0, rows_pt // ZR, zinit, 0)
            plsc.subcore_barrier()

        pltpu.async_copy(table_hbm.at[idx_all.at[0]], rows0, g0)

        def phase(j, cur, nxt, gcur, gnxt, wcur, wnxt):
            pltpu.make_async_copy(table_hbm.at[idx_all.at[j]], cur, gcur).wait()
            pltpu.async_copy(cur, out_hbm.at[pl.ds(base + j * C, C), :], wcur)
            if with_counts:
                @pl.when(j >= 1)
                def _():
                    pltpu.make_async_copy(
                        ones_v, cacc.at[ridx_all.at[0]], csem).wait()
                pltpu.async_copy(ones_v, cacc.at[ridx_all.at[j]], csem,
                                 add=True)

            @pl.when(j + 1 < iters)
            def _():
                @pl.when(j >= 1)
                def _():
                    pltpu.make_async_copy(
                        nxt, out_hbm.at[pl.ds(base, C), :], wnxt).wait()
                pltpu.async_copy(table_hbm.at[idx_all.at[j + 1]], nxt, gnxt)

        def loop_body(j, _):
            @pl.when(j % 2 == 0)
            def _():
                phase(j, rows0, rows1, g0, g1, w0, w1)

            @pl.when(j % 2 == 1)
            def _():
                phase(j, rows1, rows0, g1, g0, w1, w0)
            return 0

        lax.fori_loop(0, iters, loop_body, 0)
        pltpu.make_async_copy(rows0, out_hbm.at[pl.ds(base, C), :], w0).wait()
        pltpu.make_async_copy(rows1, out_hbm.at[pl.ds(base, C), :], w1).wait()

        if with_counts:
            pltpu.make_async_copy(ones_v, cacc.at[ridx_all.at[0]], csem).wait()
            plsc.subcore_barrier()

            def drain(j, _):
                r0 = tid * rows_pt + j * ZR
                stg = rows0.at[pl.ds(0, ZR), pl.ds(0, 16)]
                pltpu.sync_copy(cacc.at[pl.ds(r0, ZR), :], stg)

                @pl.when(cid == 0)
                def _():
                    pltpu.sync_copy(stg, cnt_a.at[pl.ds(r0, ZR), :])

                @pl.when(cid == 1)
                def _():
                    pltpu.sync_copy(stg, cnt_b.at[pl.ds(r0, ZR), :])
                return 0

            lax.fori_loop(0, rows_pt // ZR, drain, 0)

    return functools.partial(
        pl.kernel, mesh=mesh, out_type=tuple(out_type) if with_counts
        else out_type[0],
        scratch_types=scratch,
        compiler_params=pltpu.CompilerParams(use_tc_tiling_on_sc=False),
    )(body)


def _make_sc_segsum(E, N, D2):
    ew = E // NS
    C = 80
    iters = ew // C
    rows_pt = N // NS
    ZR = 25
    assert ew % C == 0 and rows_pt % ZR == 0

    mesh = plsc.VectorSubcoreMesh(core_axis_name="c", subcore_axis_name="s")

    out_type = [
        jax.ShapeDtypeStruct((N, D2), jnp.float32),
        jax.ShapeDtypeStruct((N, D2), jnp.float32),
    ]
    scratch = [
        pltpu.VMEM_SHARED((N, D2), jnp.float32),
        pltpu.VMEM((C, D2), jnp.float32),
        pltpu.VMEM((C, D2), jnp.float32),
        pltpu.VMEM((iters, C), jnp.int32),
        pltpu.SemaphoreType.DMA,
        pltpu.SemaphoreType.DMA,
        pltpu.SemaphoreType.DMA,
        pltpu.SemaphoreType.DMA,
    ]

    def body(e_lo, e_hi, recv, out_lo, out_hi,
             acc, eb0, eb1, idx_all, r0s, r1s, s0s, s1s):
        cid = lax.axis_index("c")
        tid = lax.axis_index("s")
        base = tid * ew

        pltpu.sync_copy(recv.at[tid], idx_all)

        def rstart(j, buf, sem):
            @pl.when(cid == 0)
            def _():
                pltpu.async_copy(e_lo.at[pl.ds(base + j * C, C), :], buf, sem)

            @pl.when(cid == 1)
            def _():
                pltpu.async_copy(e_hi.at[pl.ds(base + j * C, C), :], buf, sem)

        def rwait(buf, sem):
            pltpu.make_async_copy(
                e_lo.at[pl.ds(base, C), :], buf, sem).wait()

        zero16 = jnp.zeros((16,), jnp.float32)
        for r in range(ZR):
            for q in range(D2 // 16):
                eb0[r, pl.ds(q * 16, 16)] = zero16

        def zinit(j, _):
            r0 = tid * rows_pt + j * ZR
            pltpu.sync_copy(eb0.at[pl.ds(0, ZR), :], acc.at[pl.ds(r0, ZR), :])
            return 0

        lax.fori_loop(0, rows_pt // ZR, zinit, 0)
        plsc.subcore_barrier()
        rstart(0, eb0, r0s)

        def swait(buf, sem):
            pltpu.make_async_copy(buf, acc.at[idx_all.at[0]], sem).wait()

        def phase(j, cur, nxt, rcur, rnxt, scur, snxt):
            rwait(cur, rcur)
            pltpu.async_copy(cur, acc.at[idx_all.at[j]], scur, add=True)

            @pl.when(j + 1 < iters)
            def _():
                @pl.when(j >= 1)
                def _():
                    swait(nxt, snxt)
                rstart(j + 1, nxt, rnxt)

        def chunk(j, _):
            @pl.when(j % 2 == 0)
            def _():
                phase(j, eb0, eb1, r0s, r1s, s0s, s1s)

            @pl.when(j % 2 == 1)
            def _():
                phase(j, eb1, eb0, r1s, r0s, s1s, s0s)
            return 0

        lax.fori_loop(0, iters, chunk, 0)
        swait(eb0, s0s)
        swait(eb1, s1s)
        plsc.subcore_barrier()

        def drain(j, _):
            r0 = tid * rows_pt + j * ZR
            pltpu.sync_copy(acc.at[pl.ds(r0, ZR), :], eb0.at[pl.ds(0, ZR), :])

            @pl.when(cid == 0)
            def _():
                pltpu.sync_copy(eb0.at[pl.ds(0, ZR), :],
                                out_lo.at[pl.ds(r0, ZR), :])

            @pl.when(cid == 1)
            def _():
                pltpu.sync_copy(eb0.at[pl.ds(0, ZR), :],
                                out_hi.at[pl.ds(r0, ZR), :])
            return 0

        lax.fori_loop(0, rows_pt // ZR, drain, 0)

    return functools.partial(
        pl.kernel, mesh=mesh, out_type=tuple(out_type),
        scratch_types=scratch,
        compiler_params=pltpu.CompilerParams(use_tc_tiling_on_sc=False),
    )(body)


def _make_sc_segsum_esplit(E, N, D):
    ew = E // NW
    C = 80
    iters = ew // C
    rows_pt = N // NS
    ZR = 25
    assert ew % C == 0 and rows_pt % ZR == 0

    mesh = plsc.VectorSubcoreMesh(core_axis_name="c", subcore_axis_name="s")

    out_type = [
        jax.ShapeDtypeStruct((N, D), jnp.float32),
        jax.ShapeDtypeStruct((N, D), jnp.float32),
    ]
    scratch = [
        pltpu.VMEM_SHARED((N, D), jnp.float32),
        pltpu.VMEM((C, D), jnp.float32),
        pltpu.VMEM((C, D), jnp.float32),
        pltpu.VMEM((iters, C), jnp.int32),
        pltpu.SemaphoreType.DMA,
        pltpu.SemaphoreType.DMA,
        pltpu.SemaphoreType.DMA,
        pltpu.SemaphoreType.DMA,
    ]

    def body(e_hbm, recv, out_a, out_b,
             acc, eb0, eb1, idx_all, r0s, r1s, s0s, s1s):
        cid = lax.axis_index("c")
        tid = lax.axis_index("s")
        wid = tid * NC + cid
        base = wid * ew

        pltpu.sync_copy(recv.at[wid], idx_all)

        zero16 = jnp.zeros((16,), jnp.float32)
        for r in range(ZR):
            for q in range(D // 16):
                eb0[r, pl.ds(q * 16, 16)] = zero16

        def zinit(j, _):
            r0 = tid * rows_pt + j * ZR
            pltpu.sync_copy(eb0.at[pl.ds(0, ZR), :], acc.at[pl.ds(r0, ZR), :])
            return 0

        lax.fori_loop(0, rows_pt // ZR, zinit, 0)
        plsc.subcore_barrier()

        def rstart(j, buf, sem):
            pltpu.async_copy(e_hbm.at[pl.ds(base + j * C, C), :], buf, sem)

        def rwait(buf, sem):
            pltpu.make_async_copy(e_hbm.at[pl.ds(base, C), :], buf, sem).wait()

        def swait(buf, sem):
            pltpu.make_async_copy(buf, acc.at[idx_all.at[0]], sem).wait()

        rstart(0, eb0, r0s)

        def phase(j, cur, nxt, rcur, rnxt, scur, snxt):
            rwait(cur, rcur)
            pltpu.async_copy(cur, acc.at[idx_all.at[j]], scur, add=True)

            @pl.when(j + 1 < iters)
            def _():
                @pl.when(j >= 1)
                def _():
                    swait(nxt, snxt)
                rstart(j + 1, nxt, rnxt)

        def chunk(j, _):
            @pl.when(j % 2 == 0)
            def _():
                phase(j, eb0, eb1, r0s, r1s, s0s, s1s)

            @pl.when(j % 2 == 1)
            def _():
                phase(j, eb1, eb0, r1s, r0s, s1s, s0s)
            return 0

        lax.fori_loop(0, iters, chunk, 0)
        swait(eb0, s0s)
        swait(eb1, s1s)
        plsc.subcore_barrier()

        def drain(j, _):
            r0 = tid * rows_pt + j * ZR
            pltpu.sync_copy(acc.at[pl.ds(r0, ZR), :], eb0.at[pl.ds(0, ZR), :])

            @pl.when(cid == 0)
            def _():
                pltpu.sync_copy(eb0.at[pl.ds(0, ZR), :],
                                out_a.at[pl.ds(r0, ZR), :])

            @pl.when(cid == 1)
            def _():
                pltpu.sync_copy(eb0.at[pl.ds(0, ZR), :],
                                out_b.at[pl.ds(r0, ZR), :])
            return 0

        lax.fori_loop(0, rows_pt // ZR, drain, 0)

    return functools.partial(
        pl.kernel, mesh=mesh, out_type=tuple(out_type),
        scratch_types=scratch,
        compiler_params=pltpu.CompilerParams(use_tc_tiling_on_sc=False),
    )(body)


_BE = 8000


def _edge1_body(ea_ref, xs_ref, we_ref, ws_ref, b_ref, lo_ref, hi_ref):
    acc = jnp.dot(ea_ref[...], we_ref[...], preferred_element_type=jnp.float32)
    acc += jnp.dot(xs_ref[...], ws_ref[...], preferred_element_type=jnp.float32)
    e = jnp.maximum(acc + b_ref[...], 0.0)
    lo_ref[...] = e[:, :128]
    hi_ref[...] = e[:, 128:]


def _tc_edge1(edge_attr, xs, We1_e, We1_s_pad, be1):
    grid = _E // _BE
    return pl.pallas_call(
        _edge1_body,
        grid=(grid,),
        in_specs=[
            pl.BlockSpec((_BE, 10), lambda i: (i, 0)),
            pl.BlockSpec((_BE, 128), lambda i: (i, 0)),
            pl.BlockSpec((10, 256), lambda i: (0, 0)),
            pl.BlockSpec((128, 256), lambda i: (0, 0)),
            pl.BlockSpec((1, 256), lambda i: (0, 0)),
        ],
        out_specs=[
            pl.BlockSpec((_BE, 128), lambda i: (i, 0)),
            pl.BlockSpec((_BE, 128), lambda i: (i, 0)),
        ],
        out_shape=[
            jax.ShapeDtypeStruct((_E, 128), jnp.float32),
            jax.ShapeDtypeStruct((_E, 128), jnp.float32),
        ],
    )(edge_attr, xs, We1_e, We1_s_pad, be1.reshape(1, 256))


def _edge2_body(lo_ref, hi_ref, g_ref, w_ref, b_ref, out_ref):
    acc = jnp.dot(lo_ref[...], w_ref[:128, :], preferred_element_type=jnp.float32)
    acc += jnp.dot(hi_ref[...], w_ref[128:, :], preferred_element_type=jnp.float32)
    out_ref[...] = jnp.maximum(acc + g_ref[...] + b_ref[...], 0.0)


def _tc_edge2(e1_lo, e1_hi, g2, We2_e, be2):
    grid = _E // _BE
    return pl.pallas_call(
        _edge2_body,
        grid=(grid,),
        in_specs=[
            pl.BlockSpec((_BE, 128), lambda i: (i, 0)),
            pl.BlockSpec((_BE, 128), lambda i: (i, 0)),
            pl.BlockSpec((_BE, 128), lambda i: (i, 0)),
            pl.BlockSpec((256, 128), lambda i: (0, 0)),
            pl.BlockSpec((1, 128), lambda i: (0, 0)),
        ],
        out_specs=pl.BlockSpec((_BE, 128), lambda i: (i, 0)),
        out_shape=jax.ShapeDtypeStruct((_E, 128), jnp.float32),
    )(e1_lo, e1_hi, g2, We2_e, be2.reshape(1, 128))


def _edge3_body(e2_ref, g_ref, w_ref, b_ref, out_ref, sum_ref):
    i = pl.program_id(0)
    acc = jnp.dot(e2_ref[...], w_ref[...], preferred_element_type=jnp.float32)
    e = jnp.maximum(acc + g_ref[:, :64] + b_ref[...], 0.0)
    out_ref[...] = jnp.concatenate([e, jnp.zeros_like(e)], axis=1)

    @pl.when(i == 0)
    def _():
        sum_ref[...] = jnp.zeros_like(sum_ref)

    sum_ref[...] += jnp.sum(e.reshape(_BE // 8, 8, 64), axis=0)


def _tc_edge3(e2, g3, We3_e, be3):
    grid = _E // _BE
    return pl.pallas_call(
        _edge3_body,
        grid=(grid,),
        in_specs=[
            pl.BlockSpec((_BE, 128), lambda i: (i, 0)),
            pl.BlockSpec((_BE, 128), lambda i: (i, 0)),
            pl.BlockSpec((128, 64), lambda i: (0, 0)),
            pl.BlockSpec((1, 64), lambda i: (0, 0)),
        ],
        out_specs=[
            pl.BlockSpec((_BE, 128), lambda i: (i, 0)),
            pl.BlockSpec((8, 64), lambda i: (0, 0)),
        ],
        out_shape=[
            jax.ShapeDtypeStruct((_E, 128), jnp.float32),
            jax.ShapeDtypeStruct((8, 64), jnp.float32),
        ],
    )(e2, g3, We3_e, be3.reshape(1, 64))


_BN = 1000


def _node_body(prev_ref, slo_ref, shi_ref, ca_ref, cb_ref, wn_ref, wi_ref,
               b_ref, ws_ref, n_ref, m_ref):
    cnt = jnp.maximum(ca_ref[:, 0:1] + cb_ref[:, 0:1], 1.0)
    s = jnp.concatenate([slo_ref[...], shi_ref[...]], axis=1) / cnt
    acc = jnp.dot(prev_ref[...], wn_ref[...], preferred_element_type=jnp.float32)
    acc += jnp.dot(s, wi_ref[...], preferred_element_type=jnp.float32)
    n = jnp.maximum(acc + b_ref[...], 0.0)
    n_ref[...] = n
    m_ref[...] = jnp.dot(n, ws_ref[...], preferred_element_type=jnp.float32)


def _tc_node(prev, s_lo, s_hi, cnt_a, cnt_b, Wn, Wi, b, Ws):
    d_prev = prev.shape[1]
    d2 = s_lo.shape[1]
    d_out = Wn.shape[1]
    d_m = Ws.shape[1]
    grid = _N // _BN
    return pl.pallas_call(
        _node_body,
        grid=(grid,),
        in_specs=[
            pl.BlockSpec((_BN, d_prev), lambda i: (i, 0)),
            pl.BlockSpec((_BN, d2), lambda i: (i, 0)),
            pl.BlockSpec((_BN, d2), lambda i: (i, 0)),
            pl.BlockSpec((_BN, 16), lambda i: (i, 0)),
            pl.BlockSpec((_BN, 16), lambda i: (i, 0)),
            pl.BlockSpec((d_prev, d_out), lambda i: (0, 0)),
            pl.BlockSpec((2 * d2, d_out), lambda i: (0, 0)),
            pl.BlockSpec((1, d_out), lambda i: (0, 0)),
            pl.BlockSpec((d_out, d_m), lambda i: (0, 0)),
        ],
        out_specs=[
            pl.BlockSpec((_BN, d_out), lambda i: (i, 0)),
            pl.BlockSpec((_BN, d_m), lambda i: (i, 0)),
        ],
        out_shape=[
            jax.ShapeDtypeStruct((_N, d_out), jnp.float32),
            jax.ShapeDtypeStruct((_N, d_m), jnp.float32),
        ],
    )(prev, s_lo, s_hi, cnt_a, cnt_b, Wn, Wi, b.reshape(1, d_out), Ws)


def _node_sum_body(prev_ref, sa_ref, sb_ref, ca_ref, cb_ref, wn_ref, wi_ref,
                   b_ref, ws_ref, n_ref, m_ref):
    cnt = jnp.maximum(ca_ref[:, 0:1] + cb_ref[:, 0:1], 1.0)
    s = (sa_ref[...] + sb_ref[...]) / cnt
    acc = jnp.dot(prev_ref[...], wn_ref[...], preferred_element_type=jnp.float32)
    acc += jnp.dot(s, wi_ref[...], preferred_element_type=jnp.float32)
    n = jnp.maximum(acc + b_ref[...], 0.0)
    n_ref[...] = n
    m_ref[...] = jnp.dot(n, ws_ref[...], preferred_element_type=jnp.float32)


def _tc_node_sum(prev, s_a, s_b, cnt_a, cnt_b, Wn, Wi, b, Ws):
    d_prev = prev.shape[1]
    d_s = s_a.shape[1]
    d_out = Wn.shape[1]
    d_m = Ws.shape[1]
    grid = _N // _BN
    return pl.pallas_call(
        _node_sum_body,
        grid=(grid,),
        in_specs=[
            pl.BlockSpec((_BN, d_prev), lambda i: (i, 0)),
            pl.BlockSpec((_BN, d_s), lambda i: (i, 0)),
            pl.BlockSpec((_BN, d_s), lambda i: (i, 0)),
            pl.BlockSpec((_BN, 16), lambda i: (i, 0)),
            pl.BlockSpec((_BN, 16), lambda i: (i, 0)),
            pl.BlockSpec((d_prev, d_out), lambda i: (0, 0)),
            pl.BlockSpec((d_s, d_out), lambda i: (0, 0)),
            pl.BlockSpec((1, d_out), lambda i: (0, 0)),
            pl.BlockSpec((d_out, d_m), lambda i: (0, 0)),
        ],
        out_specs=[
            pl.BlockSpec((_BN, d_out), lambda i: (i, 0)),
            pl.BlockSpec((_BN, d_m), lambda i: (i, 0)),
        ],
        out_shape=[
            jax.ShapeDtypeStruct((_N, d_out), jnp.float32),
            jax.ShapeDtypeStruct((_N, d_m), jnp.float32),
        ],
    )(prev, s_a, s_b, cnt_a, cnt_b, Wn, Wi, b.reshape(1, d_out), Ws)


def _node3_body(prev_ref, sa_ref, sb_ref, ca_ref, cb_ref, esum_ref, wn_ref,
                wi_ref, b_ref, wgn_ref, wge_ref, bg_ref, g_ref, acc_ref):
    i = pl.program_id(0)
    cnt = jnp.maximum(ca_ref[:, 0:1] + cb_ref[:, 0:1], 1.0)
    s = (sa_ref[...] + sb_ref[...])[:, :64] / cnt
    acc = jnp.dot(prev_ref[...], wn_ref[...], preferred_element_type=jnp.float32)
    acc += jnp.dot(s, wi_ref[...], preferred_element_type=jnp.float32)
    n3 = jnp.maximum(acc + b_ref[...], 0.0)

    @pl.when(i == 0)
    def _():
        acc_ref[...] = jnp.zeros_like(acc_ref)

    acc_ref[...] += jnp.sum(n3.reshape(_BN // 8, 8, 64), axis=0)

    nmean = jnp.sum(acc_ref[...], axis=0, keepdims=True) * (1.0 / _N)
    emean = jnp.sum(esum_ref[...], axis=0, keepdims=True) * (1.0 / _E)
    g = jnp.dot(nmean, wgn_ref[...], preferred_element_type=jnp.float32)
    g += jnp.dot(emean, wge_ref[...], preferred_element_type=jnp.float32)
    g_ref[...] = g + bg_ref[...]


def _tc_node3(n2, s_lo, s_hi, cnt_a, cnt_b, e3sum, Wn3_n, Wn3_i, bn3,
              Wg_n, Wg_e, bg):
    grid = _N // _BN
    return pl.pallas_call(
        _node3_body,
        grid=(grid,),
        in_specs=[
            pl.BlockSpec((_BN, 128), lambda i: (i, 0)),
            pl.BlockSpec((_BN, 128), lambda i: (i, 0)),
            pl.BlockSpec((_BN, 128), lambda i: (i, 0)),
            pl.BlockSpec((_BN, 16), lambda i: (i, 0)),
            pl.BlockSpec((_BN, 16), lambda i: (i, 0)),
            pl.BlockSpec((8, 64), lambda i: (0, 0)),
            pl.BlockSpec((128, 64), lambda i: (0, 0)),
            pl.BlockSpec((64, 64), lambda i: (0, 0)),
            pl.BlockSpec((1, 64), lambda i: (0, 0)),
            pl.BlockSpec((64, 128), lambda i: (0, 0)),
            pl.BlockSpec((64, 128), lambda i: (0, 0)),
            pl.BlockSpec((1, 128), lambda i: (0, 0)),
        ],
        out_specs=pl.BlockSpec((1, 128), lambda i: (0, 0)),
        out_shape=jax.ShapeDtypeStruct((1, 128), jnp.float32),
        scratch_shapes=[pltpu.VMEM((8, 64), jnp.float32)],
    )(n2, s_lo, s_hi, cnt_a, cnt_b, e3sum, Wn3_n, Wn3_i, bn3.reshape(1, 64),
      Wg_n, Wg_e, bg.reshape(1, 128))


_sc_gather128c = _make_sc_gather(_N, 128, _E, with_counts=True)
_sc_gather128 = _make_sc_gather(_N, 128, _E)
_sc_segsum128 = _make_sc_segsum(_E, _N, 128)
_sc_segsum_es = _make_sc_segsum_esplit(_E, _N, 128)


def kernel(x, edge_attr, edge_index, We1_e, We1_s, be1, Wn1_n, Wn1_i, bn1,
           We2_e, We2_s, be2, Wn2_n, Wn2_i, bn2, We3_e, We3_s, be3,
           Wn3_n, Wn3_i, bn3, Wg_n, Wg_e, bg):
    senders = edge_index[0].reshape(NW, _E // NW // 80, 80)
    recv_nw = edge_index[1].reshape(NW, _E // NW // 80, 80)
    recv_ns = edge_index[1].reshape(NS, _E // NS // 80, 80)
    x_pad = jnp.pad(x, ((0, 0), (0, 114)))
    We1_s_pad = jnp.pad(We1_s, ((0, 114), (0, 0)))
    We3_s_pad = jnp.pad(We3_s, ((0, 0), (0, 64)))

    xs, cnt_a, cnt_b = _sc_gather128c(x_pad, senders, recv_nw)
    e1_lo, e1_hi = _tc_edge1(edge_attr, xs, We1_e, We1_s_pad, be1)
    s1_lo, s1_hi = _sc_segsum128(e1_lo, e1_hi, recv_ns)
    n1, m1 = _tc_node(x, s1_lo, s1_hi, cnt_a, cnt_b, Wn1_n, Wn1_i, bn1, We2_s)

    g2 = _sc_gather128(m1, senders)
    e2 = _tc_edge2(e1_lo, e1_hi, g2, We2_e, be2)
    s2a, s2b = _sc_segsum_es(e2, recv_nw)
    n2, m2 = _tc_node_sum(n1, s2a, s2b, cnt_a, cnt_b, Wn2_n, Wn2_i, bn2,
                          We3_s_pad)

    g3 = _sc_gather128(m2, senders)
    e3p, e3sum = _tc_edge3(e2, g3, We3_e, be3)
    s3a, s3b = _sc_segsum_es(e3p, recv_nw)
    g = _tc_node3(n2, s3a, s3b, cnt_a, cnt_b, e3sum, Wn3_n, Wn3_i, bn3,
                  Wg_n, Wg_e, bg)
    return g.reshape(128)
